# Initial kernel scaffold; baseline (speedup 1.0000x reference)
import functools

import jax
import jax.numpy as jnp
from jax import lax
from jax.experimental import pallas as pl
from jax.experimental.pallas import tpu as pltpu
from jax.experimental.pallas import tpu_sc as plsc

N = 100000
D = 128
F = 16
E = 3200000

NC, NS, LN = 2, 16, 16
B = 128
NPAD = 100352
ROWS_PT = NPAD // NS
RCH = ROWS_PT // B
TB = 100096
NBLK = TB // B
EPAD = TB * NC * NS
NEG = 0.01
R = 2000


def _dense1_body(x_ref, w_ref, a_ref, h_ref, sc_ref, amax_ref):
    i = pl.program_id(0)
    h = jnp.dot(x_ref[...], w_ref[...], preferred_element_type=jnp.float32)
    h_ref[...] = h
    a = a_ref[...]
    s_col = jnp.dot(h, a[:F, :], preferred_element_type=jnp.float32)
    d_col = jnp.dot(h, a[F:, :], preferred_element_type=jnp.float32)
    sc_ref[...] = jnp.concatenate([s_col, d_col], axis=1)
    bmax = jnp.max(d_col)

    @pl.when(i == 0)
    def _():
        amax_ref[0, 0] = bmax

    @pl.when(i > 0)
    def _():
        amax_ref[0, 0] = jnp.maximum(amax_ref[0, 0], bmax)


def _dense1(x, w, a, d_in):
    return pl.pallas_call(
        _dense1_body,
        grid=(N // R,),
        in_specs=[
            pl.BlockSpec((R, d_in), lambda i: (i, 0)),
            pl.BlockSpec((d_in, F), lambda i: (0, 0)),
            pl.BlockSpec((2 * F, 1), lambda i: (0, 0)),
        ],
        out_specs=[
            pl.BlockSpec((R, F), lambda i: (i, 0)),
            pl.BlockSpec((R, 2), lambda i: (i, 0)),
            pl.BlockSpec((1, 1), lambda i: (0, 0)),
        ],
        out_shape=[
            jax.ShapeDtypeStruct((N, F), jnp.float32),
            jax.ShapeDtypeStruct((N, 2), jnp.float32),
            jax.ShapeDtypeStruct((1, 1), jnp.float32),
        ],
        compiler_params=pltpu.CompilerParams(
            dimension_semantics=("arbitrary",)),
    )(x, w, a)


def _dense2_body(o0_ref, o1_ref, s0_ref, s1_ref, w_ref, a_ref,
                 h_ref, sc_ref, amax_ref):
    i = pl.program_id(0)
    num = o0_ref[...] + o1_ref[...]
    s = s0_ref[...] + s1_ref[...]
    x = jnp.where(s > 0.0, num / s, 0.0)
    x = jnp.where(x > 0.0, x, jnp.exp(jnp.minimum(x, 0.0)) - 1.0)
    h = jnp.dot(x, w_ref[...], preferred_element_type=jnp.float32)
    h_ref[...] = h
    a = a_ref[...]
    s_col = jnp.dot(h, a[:F, :], preferred_element_type=jnp.float32)
    d_col = jnp.dot(h, a[F:, :], preferred_element_type=jnp.float32)
    sc_ref[...] = jnp.concatenate([s_col, d_col], axis=1)
    bmax = jnp.max(d_col)

    @pl.when(i == 0)
    def _():
        amax_ref[0, 0] = bmax

    @pl.when(i > 0)
    def _():
        amax_ref[0, 0] = jnp.maximum(amax_ref[0, 0], bmax)


def _dense2(o0, o1, s0, s1, w, a):
    return pl.pallas_call(
        _dense2_body,
        grid=(N // R,),
        in_specs=[
            pl.BlockSpec((R, F), lambda i: (i, 0)),
            pl.BlockSpec((R, F), lambda i: (i, 0)),
            pl.BlockSpec((R, 1), lambda i: (i, 0)),
            pl.BlockSpec((R, 1), lambda i: (i, 0)),
            pl.BlockSpec((F, F), lambda i: (0, 0)),
            pl.BlockSpec((2 * F, 1), lambda i: (0, 0)),
        ],
        out_specs=[
            pl.BlockSpec((R, F), lambda i: (i, 0)),
            pl.BlockSpec((R, 2), lambda i: (i, 0)),
            pl.BlockSpec((1, 1), lambda i: (0, 0)),
        ],
        out_shape=[
            jax.ShapeDtypeStruct((N, F), jnp.float32),
            jax.ShapeDtypeStruct((N, 2), jnp.float32),
            jax.ShapeDtypeStruct((1, 1), jnp.float32),
        ],
        compiler_params=pltpu.CompilerParams(
            dimension_semantics=("arbitrary",)),
    )(o0, o1, s0, s1, w, a)


def _final_body(o0_ref, o1_ref, s0_ref, s1_ref, y_ref):
    num = o0_ref[...] + o1_ref[...]
    s = s0_ref[...] + s1_ref[...]
    z = jnp.where(s > 0.0, num / s, 0.0)
    m = jnp.max(z, axis=1, keepdims=True)
    zs = z - m
    y_ref[...] = zs - jnp.log(jnp.sum(jnp.exp(zs), axis=1, keepdims=True))


def _final(o0, o1, s0, s1):
    return pl.pallas_call(
        _final_body,
        grid=(N // R,),
        in_specs=[
            pl.BlockSpec((R, F), lambda i: (i, 0)),
            pl.BlockSpec((R, F), lambda i: (i, 0)),
            pl.BlockSpec((R, 1), lambda i: (i, 0)),
            pl.BlockSpec((R, 1), lambda i: (i, 0)),
        ],
        out_specs=pl.BlockSpec((R, F), lambda i: (i, 0)),
        out_shape=jax.ShapeDtypeStruct((N, F), jnp.float32),
        compiler_params=pltpu.CompilerParams(
            dimension_semantics=("arbitrary",)),
    )(o0, o1, s0, s1)


_MESH = plsc.VectorSubcoreMesh(
    core_axis_name="c", subcore_axis_name="s", num_cores=NC, num_subcores=NS)


@functools.partial(
    pl.kernel,
    out_type=(
        jax.ShapeDtypeStruct((NC, NPAD, F), jnp.float32),
        jax.ShapeDtypeStruct((NC, NPAD), jnp.float32),
    ),
    mesh=_MESH,
    scratch_types=[
        pltpu.VMEM_SHARED((NPAD, F), jnp.float32),
        pltpu.VMEM_SHARED((NPAD,), jnp.float32),
        pltpu.VMEM_SHARED((NPAD,), jnp.float32),
        pltpu.VMEM_SHARED((NPAD,), jnp.float32),
        pltpu.VMEM_SHARED((NPAD,), jnp.float32),
        pltpu.VMEM((B,), jnp.int32),
        pltpu.VMEM((B,), jnp.int32),
        pltpu.VMEM((B, F), jnp.float32),
        pltpu.VMEM((B,), jnp.float32),
        pltpu.VMEM((B,), jnp.float32),
        pltpu.VMEM((B,), jnp.float32),
        pltpu.VMEM((B,), jnp.float32),
        pltpu.VMEM((B, F), jnp.float32),
        pltpu.VMEM((B,), jnp.float32),
    ],
)
def _sc_edge(src_hbm, dst_hbm, h_hbm, as_hbm, c_hbm, ad_hbm,
             outp_hbm, sp_hbm,
             sh_out, sh_s, sh_as, sh_c, sh_ad,
             idx_s, idx_d, rows, asv, cv, adv, exv, zrows, zs):
    cid = lax.axis_index("c")
    sid = lax.axis_index("s")
    row0 = sid * ROWS_PT

    zero16 = jnp.zeros((LN,), jnp.float32)

    def zb(i, carry):
        zrows[i, :] = zero16
        return carry

    lax.fori_loop(0, B, zb, 0)
    for j in range(B // LN):
        zs[pl.ds(j * LN, LN)] = zero16

    def stage_chunk(k, carry):
        r0 = row0 + k * B
        sl = pl.ds(r0, B)
        pltpu.sync_copy(zrows, sh_out.at[sl])
        pltpu.sync_copy(zs, sh_s.at[sl])
        pltpu.sync_copy(as_hbm.at[sl], asv)
        pltpu.sync_copy(asv, sh_as.at[sl])
        pltpu.sync_copy(c_hbm.at[sl], cv)
        pltpu.sync_copy(cv, sh_c.at[sl])
        pltpu.sync_copy(ad_hbm.at[sl], adv)
        pltpu.sync_copy(adv, sh_ad.at[sl])
        return carry

    lax.fori_loop(0, RCH, stage_chunk, 0)
    plsc.subcore_barrier()

    tile_base = (cid * NS + sid) * TB

    def edge_body(blk, carry):
        base = tile_base + blk * B
        pltpu.sync_copy(src_hbm.at[pl.ds(base, B)], idx_s)
        pltpu.sync_copy(dst_hbm.at[pl.ds(base, B)], idx_d)
        pltpu.sync_copy(sh_as.at[idx_s], asv)
        pltpu.sync_copy(sh_c.at[idx_s], cv)
        pltpu.sync_copy(sh_ad.at[idx_d], adv)
        pltpu.sync_copy(h_hbm.at[idx_d], rows)
        for j in range(B // LN):
            sl = pl.ds(j * LN, LN)
            z = asv[sl] + adv[sl]
            e = jnp.where(z > 0.0, z, NEG * z)
            exv[sl] = jnp.exp(e - cv[sl])

        def scale(b, c2):
            rows[b, :] = rows[b, :] * exv[b]
            return c2

        lax.fori_loop(0, B, scale, 0)
        pltpu.sync_copy(exv, sh_s.at[idx_s], add=True)
        pltpu.sync_copy(rows, sh_out.at[idx_s], add=True)
        return carry

    lax.fori_loop(0, NBLK, edge_body, 0)
    plsc.subcore_barrier()

    def wb(k, carry):
        r0 = row0 + k * B
        sl = pl.ds(r0, B)
        pltpu.sync_copy(sh_out.at[sl], rows)
        pltpu.sync_copy(rows, outp_hbm.at[cid, sl])
        pltpu.sync_copy(sh_s.at[sl], exv)
        pltpu.sync_copy(exv, sp_hbm.at[cid, sl])
        return carry

    lax.fori_loop(0, RCH, wb, 0)


def _sc_pass(h, a_s, c, a_d, srcp, dstp):
    hp = jnp.pad(h, ((0, NPAD - N), (0, 0)))
    asp = jnp.pad(a_s, (0, NPAD - N))
    cp = jnp.pad(c, (0, NPAD - N))
    adp = jnp.pad(a_d, (0, NPAD - N))
    outp, sp = _sc_edge(srcp, dstp, hp, asp, cp, adp)
    o0 = outp[0, :N, :]
    o1 = outp[1, :N, :]
    s0 = sp[0, :N, None]
    s1 = sp[1, :N, None]
    return o0, o1, s0, s1


def kernel(local_features, edge_index, W0, a0, W1, a1):
    src = edge_index[0, :]
    dst = edge_index[1, :]
    srcp = jnp.concatenate([src, jnp.full((EPAD - E,), N, jnp.int32)])
    dstp = jnp.concatenate([dst, jnp.full((EPAD - E,), N, jnp.int32)])

    h1, sc1, amax1 = _dense1(local_features, W0, a0, D)
    as1 = sc1[:, 0]
    ad1 = sc1[:, 1]
    z1 = as1 + amax1[0, 0]
    c1 = jnp.where(z1 > 0.0, z1, NEG * z1)
    o0, o1, s0, s1 = _sc_pass(h1, as1, c1, ad1, srcp, dstp)

    h2, sc2, amax2 = _dense2(o0, o1, s0, s1, W1, a1)
    as2 = sc2[:, 0]
    ad2 = sc2[:, 1]
    z2 = as2 + amax2[0, 0]
    c2 = jnp.where(z2 > 0.0, z2, NEG * z2)
    o0, o1, s0, s1 = _sc_pass(h2, as2, c2, ad2, srcp, dstp)

    return _final(o0, o1, s0, s1)


# speedup vs baseline: 18.9014x; 18.9014x over previous
import functools

import jax
import jax.numpy as jnp
from jax import lax
from jax.experimental import pallas as pl
from jax.experimental.pallas import tpu as pltpu
from jax.experimental.pallas import tpu_sc as plsc

N = 100000
D = 128
F = 16
E = 3200000

NC, NS, LN = 2, 16, 16
B = 128
NPAD = 100352
ROWS_PT = NPAD // NS
RCH = ROWS_PT // B
TB = 100096
NBLK = TB // B
EPAD = TB * NC * NS
NEG = 0.01
KT2 = 4.0
R = 2000


def _lrelu(z):
    return jnp.where(z > 0.0, z, NEG * z)


def _dense_math(h, a_ref, i, sc_ref, st_ref):
    a = a_ref[...]
    s_col = jnp.dot(h, a[:F, :], preferred_element_type=jnp.float32)
    d_col = jnp.dot(h, a[F:, :], preferred_element_type=jnp.float32)
    sc_ref[...] = jnp.concatenate([s_col, d_col], axis=1)
    mx_s, mn_s = jnp.max(s_col), jnp.min(s_col)
    mx_d, mn_d = jnp.max(d_col), jnp.min(d_col)

    @pl.when(i == 0)
    def _():
        st_ref[0, 0] = mx_s
        st_ref[0, 1] = mn_s
        st_ref[0, 2] = mx_d
        st_ref[0, 3] = mn_d

    @pl.when(i > 0)
    def _():
        st_ref[0, 0] = jnp.maximum(st_ref[0, 0], mx_s)
        st_ref[0, 1] = jnp.minimum(st_ref[0, 1], mn_s)
        st_ref[0, 2] = jnp.maximum(st_ref[0, 2], mx_d)
        st_ref[0, 3] = jnp.minimum(st_ref[0, 3], mn_d)


def _dense1_body(x_ref, w_ref, a_ref, h_ref, sc_ref, st_ref):
    h = jnp.dot(x_ref[...], w_ref[...], preferred_element_type=jnp.float32)
    h_ref[...] = h
    _dense_math(h, a_ref, pl.program_id(0), sc_ref, st_ref)


def _dense1(x, w, a):
    return pl.pallas_call(
        _dense1_body,
        grid=(N // R,),
        in_specs=[
            pl.BlockSpec((R, D), lambda i: (i, 0)),
            pl.BlockSpec((D, F), lambda i: (0, 0)),
            pl.BlockSpec((2 * F, 1), lambda i: (0, 0)),
        ],
        out_specs=[
            pl.BlockSpec((R, F), lambda i: (i, 0)),
            pl.BlockSpec((R, 2), lambda i: (i, 0)),
            pl.BlockSpec(memory_space=pltpu.SMEM),
        ],
        out_shape=[
            jax.ShapeDtypeStruct((N, F), jnp.float32),
            jax.ShapeDtypeStruct((N, 2), jnp.float32),
            jax.ShapeDtypeStruct((1, 8), jnp.float32),
        ],
        compiler_params=pltpu.CompilerParams(
            dimension_semantics=("arbitrary",)),
    )(x, w, a)


def _dense2_body(o0_ref, o1_ref, s0_ref, s1_ref, w_ref, a_ref,
                 h_ref, sc_ref, st_ref):
    num = o0_ref[...] + o1_ref[...]
    s = s0_ref[...] + s1_ref[...]
    x = jnp.where(s > 0.0, num / s, 0.0)
    x = jnp.where(x > 0.0, x, jnp.exp(jnp.minimum(x, 0.0)) - 1.0)
    h = jnp.dot(x, w_ref[...], preferred_element_type=jnp.float32)
    h_ref[...] = h
    _dense_math(h, a_ref, pl.program_id(0), sc_ref, st_ref)


def _dense2(o0, o1, s0, s1, w, a):
    return pl.pallas_call(
        _dense2_body,
        grid=(N // R,),
        in_specs=[
            pl.BlockSpec((R, F), lambda i: (i, 0)),
            pl.BlockSpec((R, F), lambda i: (i, 0)),
            pl.BlockSpec((R, 1), lambda i: (i, 0)),
            pl.BlockSpec((R, 1), lambda i: (i, 0)),
            pl.BlockSpec((F, F), lambda i: (0, 0)),
            pl.BlockSpec((2 * F, 1), lambda i: (0, 0)),
        ],
        out_specs=[
            pl.BlockSpec((R, F), lambda i: (i, 0)),
            pl.BlockSpec((R, 2), lambda i: (i, 0)),
            pl.BlockSpec(memory_space=pltpu.SMEM),
        ],
        out_shape=[
            jax.ShapeDtypeStruct((N, F), jnp.float32),
            jax.ShapeDtypeStruct((N, 2), jnp.float32),
            jax.ShapeDtypeStruct((1, 8), jnp.float32),
        ],
        compiler_params=pltpu.CompilerParams(
            dimension_semantics=("arbitrary",)),
    )(o0, o1, s0, s1, w, a)


def _shift_body(t0_ref, t1_ref, c_ref, kt_ref, out_ref):
    t = t0_ref[...] + t1_ref[...]
    c = c_ref[...]
    out_ref[...] = jnp.where(t > 0.0, c + kt_ref[0, 0] * jnp.log(t), c)


def _shift_update(t0, t1, c, kt):
    return pl.pallas_call(
        _shift_body,
        grid=(N // R,),
        in_specs=[
            pl.BlockSpec((R, 1), lambda i: (i, 0)),
            pl.BlockSpec((R, 1), lambda i: (i, 0)),
            pl.BlockSpec((R, 1), lambda i: (i, 0)),
            pl.BlockSpec((1, 1), lambda i: (0, 0)),
        ],
        out_specs=pl.BlockSpec((R, 1), lambda i: (i, 0)),
        out_shape=jax.ShapeDtypeStruct((N, 1), jnp.float32),
        compiler_params=pltpu.CompilerParams(
            dimension_semantics=("arbitrary",)),
    )(t0, t1, c, kt)


def _final_body(o0_ref, o1_ref, s0_ref, s1_ref, y_ref):
    num = o0_ref[...] + o1_ref[...]
    s = s0_ref[...] + s1_ref[...]
    z = jnp.where(s > 0.0, num / s, 0.0)
    m = jnp.max(z, axis=1, keepdims=True)
    zs = z - m
    y_ref[...] = zs - jnp.log(jnp.sum(jnp.exp(zs), axis=1, keepdims=True))


def _final(o0, o1, s0, s1):
    return pl.pallas_call(
        _final_body,
        grid=(N // R,),
        in_specs=[
            pl.BlockSpec((R, F), lambda i: (i, 0)),
            pl.BlockSpec((R, F), lambda i: (i, 0)),
            pl.BlockSpec((R, 1), lambda i: (i, 0)),
            pl.BlockSpec((R, 1), lambda i: (i, 0)),
        ],
        out_specs=pl.BlockSpec((R, F), lambda i: (i, 0)),
        out_shape=jax.ShapeDtypeStruct((N, F), jnp.float32),
        compiler_params=pltpu.CompilerParams(
            dimension_semantics=("arbitrary",)),
    )(o0, o1, s0, s1)


_MESH = plsc.VectorSubcoreMesh(
    core_axis_name="c", subcore_axis_name="s", num_cores=NC, num_subcores=NS)


def _zero_vmem(buf, n_rows):
    zero16 = jnp.zeros((LN,), jnp.float32)

    def zb(i, carry):
        buf[i, :] = zero16
        return carry

    lax.fori_loop(0, n_rows, zb, 0)


def _stage_scalar(hbm, sh, bounce, row0):
    def body(k, carry):
        sl = pl.ds(row0 + k * B, B)
        pltpu.sync_copy(hbm.at[sl], bounce)
        pltpu.sync_copy(bounce, sh.at[sl])
        return carry

    lax.fori_loop(0, RCH, body, 0)


def _zero_shared(sh, zbuf, row0):
    def body(k, carry):
        pltpu.sync_copy(zbuf, sh.at[pl.ds(row0 + k * B, B)])
        return carry

    lax.fori_loop(0, RCH, body, 0)


@functools.partial(
    pl.kernel,
    out_type=jax.ShapeDtypeStruct((NC, NPAD), jnp.float32),
    mesh=_MESH,
    compiler_params=pltpu.CompilerParams(use_tc_tiling_on_sc=False),
    scratch_types=[
        pltpu.VMEM_SHARED((NPAD,), jnp.float32),
        pltpu.VMEM_SHARED((NPAD,), jnp.float32),
        pltpu.VMEM_SHARED((NPAD,), jnp.float32),
        pltpu.VMEM_SHARED((NPAD,), jnp.float32),
        pltpu.VMEM((B,), jnp.int32),
        pltpu.VMEM((B,), jnp.int32),
        pltpu.VMEM((B,), jnp.float32),
        pltpu.VMEM((B,), jnp.float32),
        pltpu.VMEM((B,), jnp.float32),
        pltpu.VMEM((B,), jnp.float32),
        pltpu.VMEM((B,), jnp.float32),
        pltpu.VMEM((16,), jnp.float32),
    ],
)
def _sc_lse(src_hbm, dst_hbm, as_hbm, c_hbm, ad_hbm, par_hbm,
            tp_hbm,
            sh_t, sh_as, sh_c, sh_ad,
            idx_s, idx_d, asv, cv, adv, exv, zs, par):
    cid = lax.axis_index("c")
    sid = lax.axis_index("s")
    row0 = sid * ROWS_PT

    pltpu.sync_copy(par_hbm, par)
    ktinv = par[pl.ds(0, LN)][0]

    zero16 = jnp.zeros((LN,), jnp.float32)
    for j in range(B // LN):
        zs[pl.ds(j * LN, LN)] = zero16
    _zero_shared(sh_t, zs, row0)
    _stage_scalar(as_hbm, sh_as, asv, row0)
    _stage_scalar(c_hbm, sh_c, cv, row0)
    _stage_scalar(ad_hbm, sh_ad, adv, row0)
    plsc.subcore_barrier()

    tile_base = (cid * NS + sid) * TB

    def edge_body(blk, carry):
        base = tile_base + blk * B
        pltpu.sync_copy(src_hbm.at[pl.ds(base, B)], idx_s)
        pltpu.sync_copy(dst_hbm.at[pl.ds(base, B)], idx_d)
        pltpu.sync_copy(sh_as.at[idx_s], asv)
        pltpu.sync_copy(sh_c.at[idx_s], cv)
        pltpu.sync_copy(sh_ad.at[idx_d], adv)
        for j in range(B // LN):
            sl = pl.ds(j * LN, LN)
            z = asv[sl] + adv[sl]
            exv[sl] = jnp.exp((_lrelu(z) - cv[sl]) * ktinv)
        pltpu.sync_copy(exv, sh_t.at[idx_s], add=True)
        return carry

    lax.fori_loop(0, NBLK, edge_body, 0)
    plsc.subcore_barrier()

    def wb(k, carry):
        sl = pl.ds(row0 + k * B, B)
        pltpu.sync_copy(sh_t.at[sl], exv)
        pltpu.sync_copy(exv, tp_hbm.at[cid, sl])
        return carry

    lax.fori_loop(0, RCH, wb, 0)


@functools.partial(
    pl.kernel,
    out_type=(
        jax.ShapeDtypeStruct((NC, NPAD, F), jnp.float32),
        jax.ShapeDtypeStruct((NC, NPAD), jnp.float32),
    ),
    mesh=_MESH,
    compiler_params=pltpu.CompilerParams(use_tc_tiling_on_sc=False),
    scratch_types=[
        pltpu.VMEM_SHARED((NPAD, F), jnp.float32),
        pltpu.VMEM_SHARED((NPAD,), jnp.float32),
        pltpu.VMEM_SHARED((NPAD,), jnp.float32),
        pltpu.VMEM_SHARED((NPAD,), jnp.float32),
        pltpu.VMEM_SHARED((NPAD,), jnp.float32),
        pltpu.VMEM((B,), jnp.int32),
        pltpu.VMEM((B,), jnp.int32),
        pltpu.VMEM((B, F), jnp.float32),
        pltpu.VMEM((B,), jnp.float32),
        pltpu.VMEM((B,), jnp.float32),
        pltpu.VMEM((B,), jnp.float32),
        pltpu.VMEM((B,), jnp.float32),
        pltpu.VMEM((B, F), jnp.float32),
        pltpu.VMEM((B,), jnp.float32),
    ],
)
def _sc_full(src_hbm, dst_hbm, h_hbm, as_hbm, c_hbm, ad_hbm,
             outp_hbm, sp_hbm,
             sh_out, sh_s, sh_as, sh_c, sh_ad,
             idx_s, idx_d, rows, asv, cv, adv, exv, zrows, zs):
    cid = lax.axis_index("c")
    sid = lax.axis_index("s")
    row0 = sid * ROWS_PT

    _zero_vmem(zrows, B)
    zero16 = jnp.zeros((LN,), jnp.float32)
    for j in range(B // LN):
        zs[pl.ds(j * LN, LN)] = zero16

    def zchunk(k, carry):
        sl = pl.ds(row0 + k * B, B)
        pltpu.sync_copy(zrows, sh_out.at[sl])
        pltpu.sync_copy(zs, sh_s.at[sl])
        return carry

    lax.fori_loop(0, RCH, zchunk, 0)
    _stage_scalar(as_hbm, sh_as, asv, row0)
    _stage_scalar(c_hbm, sh_c, cv, row0)
    _stage_scalar(ad_hbm, sh_ad, adv, row0)
    plsc.subcore_barrier()

    tile_base = (cid * NS + sid) * TB

    def edge_body(blk, carry):
        base = tile_base + blk * B
        pltpu.sync_copy(src_hbm.at[pl.ds(base, B)], idx_s)
        pltpu.sync_copy(dst_hbm.at[pl.ds(base, B)], idx_d)
        pltpu.sync_copy(sh_as.at[idx_s], asv)
        pltpu.sync_copy(sh_c.at[idx_s], cv)
        pltpu.sync_copy(sh_ad.at[idx_d], adv)
        pltpu.sync_copy(h_hbm.at[idx_d], rows)
        for j in range(B // LN):
            sl = pl.ds(j * LN, LN)
            z = asv[sl] + adv[sl]
            ex = jnp.exp(_lrelu(z) - cv[sl])
            exv[sl] = ex
            for t in range(LN):
                b = j * LN + t
                rows[b, :] = rows[b, :] * ex[t]
        pltpu.sync_copy(exv, sh_s.at[idx_s], add=True)
        pltpu.sync_copy(rows, sh_out.at[idx_s], add=True)
        return carry

    lax.fori_loop(0, NBLK, edge_body, 0)
    plsc.subcore_barrier()

    def wb(k, carry):
        sl = pl.ds(row0 + k * B, B)
        pltpu.sync_copy(sh_out.at[sl], rows)
        pltpu.sync_copy(rows, outp_hbm.at[cid, sl])
        pltpu.sync_copy(sh_s.at[sl], exv)
        pltpu.sync_copy(exv, sp_hbm.at[cid, sl])
        return carry

    lax.fori_loop(0, RCH, wb, 0)


def _layer_edge(h, sc, st, srcp, dstp):
    a_s = sc[:, 0]
    a_d = sc[:, 1]
    hi = _lrelu(st[0, 0] + st[0, 2])
    lo = _lrelu(st[0, 1] + st[0, 3])
    kt1 = jnp.maximum(1.0, (hi - lo) * (1.0 / 80.0))

    npad = NPAD - N
    asp = jnp.pad(a_s, (0, npad))
    adp = jnp.pad(a_d, (0, npad))
    c0p = jnp.pad(jnp.full((N,), hi, jnp.float32), (0, npad))

    par1 = jnp.zeros((16,), jnp.float32).at[0].set(1.0 / kt1)
    tp = _sc_lse(srcp, dstp, asp, c0p, adp, par1)
    c1 = _shift_update(tp[0, :N, None], tp[1, :N, None],
                       c0p[:N, None], kt1[None, None])[:, 0]

    par2 = jnp.zeros((16,), jnp.float32).at[0].set(1.0 / KT2)
    c1p = jnp.pad(c1, (0, npad))
    tp2 = _sc_lse(srcp, dstp, asp, c1p, adp, par2)
    c2 = _shift_update(tp2[0, :N, None], tp2[1, :N, None],
                       c1p[:N, None],
                       jnp.full((1, 1), KT2, jnp.float32))[:, 0]

    hp = jnp.pad(h, ((0, npad), (0, 0)))
    c2p = jnp.pad(c2, (0, npad))
    outp, sp = _sc_full(srcp, dstp, hp, asp, c2p, adp)
    o0 = outp[0, :N, :]
    o1 = outp[1, :N, :]
    s0 = sp[0, :N, None]
    s1 = sp[1, :N, None]
    return o0, o1, s0, s1


def kernel(local_features, edge_index, W0, a0, W1, a1):
    src = edge_index[0, :]
    dst = edge_index[1, :]
    srcp = jnp.concatenate([src, jnp.full((EPAD - E,), N, jnp.int32)])
    dstp = jnp.concatenate([dst, jnp.full((EPAD - E,), N, jnp.int32)])

    h1, sc1, st1 = _dense1(local_features, W0, a0)
    o0, o1, s0, s1 = _layer_edge(h1, sc1, st1, srcp, dstp)

    h2, sc2, st2 = _dense2(o0, o1, s0, s1, W1, a1)
    o0, o1, s0, s1 = _layer_edge(h2, sc2, st2, srcp, dstp)

    return _final(o0, o1, s0, s1)


# speedup vs baseline: 41.5802x; 2.1998x over previous
import functools

import jax
import jax.numpy as jnp
from jax import lax
from jax.experimental import pallas as pl
from jax.experimental.pallas import tpu as pltpu
from jax.experimental.pallas import tpu_sc as plsc

N = 100000
D = 128
F = 16
E = 3200000

NC, NS, LN = 2, 16, 16
B = 128
NPAD = 100096
ROWS_PT = NPAD // NS
RCH = 48
TAIL = ROWS_PT - RCH * B
TB = 100096
NBLK = TB // B
EPAD = TB * NC * NS
NEG = 0.01
KT2 = 4.0
R = 2000


def _lrelu(z):
    return jnp.where(z > 0.0, z, NEG * z)


def _dense_math(h, a_ref, i, sc_ref, st_ref):
    a = a_ref[...]
    s_col = jnp.dot(h, a[:F, :], preferred_element_type=jnp.float32)
    d_col = jnp.dot(h, a[F:, :], preferred_element_type=jnp.float32)
    sc_ref[...] = jnp.concatenate([s_col, d_col], axis=1)
    mx_s, mn_s = jnp.max(s_col), jnp.min(s_col)
    mx_d, mn_d = jnp.max(d_col), jnp.min(d_col)

    @pl.when(i == 0)
    def _():
        st_ref[0, 0] = mx_s
        st_ref[0, 1] = mn_s
        st_ref[0, 2] = mx_d
        st_ref[0, 3] = mn_d

    @pl.when(i > 0)
    def _():
        st_ref[0, 0] = jnp.maximum(st_ref[0, 0], mx_s)
        st_ref[0, 1] = jnp.minimum(st_ref[0, 1], mn_s)
        st_ref[0, 2] = jnp.maximum(st_ref[0, 2], mx_d)
        st_ref[0, 3] = jnp.minimum(st_ref[0, 3], mn_d)


def _dense1_body(x_ref, w_ref, a_ref, h_ref, sc_ref, st_ref):
    h = jnp.dot(x_ref[...], w_ref[...], preferred_element_type=jnp.float32)
    h_ref[...] = h
    _dense_math(h, a_ref, pl.program_id(0), sc_ref, st_ref)


def _dense1(x, w, a):
    return pl.pallas_call(
        _dense1_body,
        grid=(N // R,),
        in_specs=[
            pl.BlockSpec((R, D), lambda i: (i, 0)),
            pl.BlockSpec((D, F), lambda i: (0, 0)),
            pl.BlockSpec((2 * F, 1), lambda i: (0, 0)),
        ],
        out_specs=[
            pl.BlockSpec((R, F), lambda i: (i, 0)),
            pl.BlockSpec((R, 2), lambda i: (i, 0)),
            pl.BlockSpec(memory_space=pltpu.SMEM),
        ],
        out_shape=[
            jax.ShapeDtypeStruct((N, F), jnp.float32),
            jax.ShapeDtypeStruct((N, 2), jnp.float32),
            jax.ShapeDtypeStruct((1, 8), jnp.float32),
        ],
        compiler_params=pltpu.CompilerParams(
            dimension_semantics=("arbitrary",)),
    )(x, w, a)


def _dense2_body(o0_ref, o1_ref, s0_ref, s1_ref, w_ref, a_ref,
                 h_ref, sc_ref, st_ref):
    num = o0_ref[...] + o1_ref[...]
    s = s0_ref[...] + s1_ref[...]
    x = jnp.where(s > 0.0, num / s, 0.0)
    x = jnp.where(x > 0.0, x, jnp.exp(jnp.minimum(x, 0.0)) - 1.0)
    h = jnp.dot(x, w_ref[...], preferred_element_type=jnp.float32)
    h_ref[...] = h
    _dense_math(h, a_ref, pl.program_id(0), sc_ref, st_ref)


def _dense2(o0, o1, s0, s1, w, a):
    return pl.pallas_call(
        _dense2_body,
        grid=(N // R,),
        in_specs=[
            pl.BlockSpec((R, F), lambda i: (i, 0)),
            pl.BlockSpec((R, F), lambda i: (i, 0)),
            pl.BlockSpec((R, 1), lambda i: (i, 0)),
            pl.BlockSpec((R, 1), lambda i: (i, 0)),
            pl.BlockSpec((F, F), lambda i: (0, 0)),
            pl.BlockSpec((2 * F, 1), lambda i: (0, 0)),
        ],
        out_specs=[
            pl.BlockSpec((R, F), lambda i: (i, 0)),
            pl.BlockSpec((R, 2), lambda i: (i, 0)),
            pl.BlockSpec(memory_space=pltpu.SMEM),
        ],
        out_shape=[
            jax.ShapeDtypeStruct((N, F), jnp.float32),
            jax.ShapeDtypeStruct((N, 2), jnp.float32),
            jax.ShapeDtypeStruct((1, 8), jnp.float32),
        ],
        compiler_params=pltpu.CompilerParams(
            dimension_semantics=("arbitrary",)),
    )(o0, o1, s0, s1, w, a)


def _shift_body(t0_ref, t1_ref, c_ref, kt_ref, out_ref):
    t = t0_ref[...] + t1_ref[...]
    c = c_ref[...]
    out_ref[...] = jnp.where(t > 0.0, c + kt_ref[0, 0] * jnp.log(t), c)


def _shift_update(t0, t1, c, kt):
    return pl.pallas_call(
        _shift_body,
        grid=(N // R,),
        in_specs=[
            pl.BlockSpec((R, 1), lambda i: (i, 0)),
            pl.BlockSpec((R, 1), lambda i: (i, 0)),
            pl.BlockSpec((R, 1), lambda i: (i, 0)),
            pl.BlockSpec((1, 1), lambda i: (0, 0)),
        ],
        out_specs=pl.BlockSpec((R, 1), lambda i: (i, 0)),
        out_shape=jax.ShapeDtypeStruct((N, 1), jnp.float32),
        compiler_params=pltpu.CompilerParams(
            dimension_semantics=("arbitrary",)),
    )(t0, t1, c, kt)


def _final_body(o0_ref, o1_ref, s0_ref, s1_ref, y_ref):
    num = o0_ref[...] + o1_ref[...]
    s = s0_ref[...] + s1_ref[...]
    z = jnp.where(s > 0.0, num / s, 0.0)
    m = jnp.max(z, axis=1, keepdims=True)
    zs = z - m
    y_ref[...] = zs - jnp.log(jnp.sum(jnp.exp(zs), axis=1, keepdims=True))


def _final(o0, o1, s0, s1):
    return pl.pallas_call(
        _final_body,
        grid=(N // R,),
        in_specs=[
            pl.BlockSpec((R, F), lambda i: (i, 0)),
            pl.BlockSpec((R, F), lambda i: (i, 0)),
            pl.BlockSpec((R, 1), lambda i: (i, 0)),
            pl.BlockSpec((R, 1), lambda i: (i, 0)),
        ],
        out_specs=pl.BlockSpec((R, F), lambda i: (i, 0)),
        out_shape=jax.ShapeDtypeStruct((N, F), jnp.float32),
        compiler_params=pltpu.CompilerParams(
            dimension_semantics=("arbitrary",)),
    )(o0, o1, s0, s1)


_MESH = plsc.VectorSubcoreMesh(
    core_axis_name="c", subcore_axis_name="s", num_cores=NC, num_subcores=NS)


def _zero_vmem(buf, n_rows):
    zero16 = jnp.zeros((LN,), jnp.float32)

    def zb(i, carry):
        buf[i, :] = zero16
        return carry

    lax.fori_loop(0, n_rows, zb, 0)


def _stage_scalar(hbm, sh, bounce, row0):
    def body(k, carry):
        sl = pl.ds(row0 + k * B, B)
        pltpu.sync_copy(hbm.at[sl], bounce)
        pltpu.sync_copy(bounce, sh.at[sl])
        return carry

    lax.fori_loop(0, RCH, body, 0)
    sl = pl.ds(row0 + RCH * B, TAIL)
    pltpu.sync_copy(hbm.at[sl], bounce.at[pl.ds(0, TAIL)])
    pltpu.sync_copy(bounce.at[pl.ds(0, TAIL)], sh.at[sl])


def _zero_shared(sh, zbuf, row0):
    def body(k, carry):
        pltpu.sync_copy(zbuf, sh.at[pl.ds(row0 + k * B, B)])
        return carry

    lax.fori_loop(0, RCH, body, 0)
    pltpu.sync_copy(zbuf.at[pl.ds(0, TAIL)],
                    sh.at[pl.ds(row0 + RCH * B, TAIL)])

def _make_lse(write_p):
    out_type = (jax.ShapeDtypeStruct((NC, NPAD), jnp.float32),)
    if write_p:
        out_type = out_type + (jax.ShapeDtypeStruct((EPAD,), jnp.float32),)

    @functools.partial(
        pl.kernel,
        out_type=out_type,
        mesh=_MESH,
        compiler_params=pltpu.CompilerParams(use_tc_tiling_on_sc=False),
        scratch_types=[
            pltpu.VMEM_SHARED((NPAD,), jnp.float32),
            pltpu.VMEM_SHARED((NPAD,), jnp.float32),
            pltpu.VMEM_SHARED((NPAD,), jnp.float32),
            pltpu.VMEM_SHARED((NPAD,), jnp.float32),
            [pltpu.VMEM((B,), jnp.int32)] * 2,
            [pltpu.VMEM((B,), jnp.int32)] * 2,
            [pltpu.VMEM((B,), jnp.float32)] * 2,
            [pltpu.VMEM((B,), jnp.float32)] * 2,
            [pltpu.VMEM((B,), jnp.float32)] * 2,
            [pltpu.VMEM((B,), jnp.int32)] * 2,
            [pltpu.VMEM((B,), jnp.float32)] * 2,
            [pltpu.SemaphoreType.DMA] * 2,
            [pltpu.SemaphoreType.DMA] * 2,
            [pltpu.SemaphoreType.DMA] * 2,
            [pltpu.SemaphoreType.DMA] * 2,
            [pltpu.SemaphoreType.DMA] * 2,
            [pltpu.SemaphoreType.DMA] * 2,
            [pltpu.SemaphoreType.DMA] * 2,
            pltpu.VMEM((16,), jnp.float32),
        ],
    )
    def lse_kernel(src_hbm, dst_hbm, as_hbm, c_hbm, ad_hbm, par_hbm, *rest):
        if write_p:
            (tp_hbm, p_hbm) = rest[0], rest[1]
            (sh_t, sh_as, sh_c, sh_ad, IS, ID, AS, CV, AD, SIDX, SEX,
             SIS, SID, SG1, SG2, SG3, SW1, SW2, par) = rest[2:]
        else:
            tp_hbm = rest[0]
            p_hbm = None
            (sh_t, sh_as, sh_c, sh_ad, IS, ID, AS, CV, AD, SIDX, SEX,
             SIS, SID, SG1, SG2, SG3, SW1, SW2, par) = rest[1:]

        cid = lax.axis_index("c")
        sid = lax.axis_index("s")
        row0 = sid * ROWS_PT

        pltpu.sync_copy(par_hbm, par)
        ktinv = par[pl.ds(0, LN)][0]

        zero16 = jnp.zeros((LN,), jnp.float32)
        for j in range(B // LN):
            SEX[0][pl.ds(j * LN, LN)] = zero16
        _zero_shared(sh_t, SEX[0], row0)
        _stage_scalar(as_hbm, sh_as, AS[0], row0)
        _stage_scalar(c_hbm, sh_c, CV[0], row0)
        _stage_scalar(ad_hbm, sh_ad, AD[0], row0)
        plsc.subcore_barrier()

        tile_base = (cid * NS + sid) * TB

        def issue_idx(b, blk):
            base = tile_base + blk * B
            pltpu.async_copy(src_hbm.at[pl.ds(base, B)], IS[b], SIS[b])
            pltpu.async_copy(dst_hbm.at[pl.ds(base, B)], ID[b], SID[b])

        def wait_idx(b, blk):
            base = tile_base + blk * B
            pltpu.make_async_copy(src_hbm.at[pl.ds(base, B)], IS[b],
                                  SIS[b]).wait()
            pltpu.make_async_copy(dst_hbm.at[pl.ds(base, B)], ID[b],
                                  SID[b]).wait()

        def issue_gath(b):
            pltpu.async_copy(sh_as.at[IS[b]], AS[b], SG1[b])
            pltpu.async_copy(sh_c.at[IS[b]], CV[b], SG2[b])
            pltpu.async_copy(sh_ad.at[ID[b]], AD[b], SG3[b])

        def wait_gath(b):
            pltpu.make_async_copy(sh_as.at[IS[b]], AS[b], SG1[b]).wait()
            pltpu.make_async_copy(sh_c.at[IS[b]], CV[b], SG2[b]).wait()
            pltpu.make_async_copy(sh_ad.at[ID[b]], AD[b], SG3[b]).wait()

        def compute(b):
            for j in range(B // LN):
                sl = pl.ds(j * LN, LN)
                z = AS[b][sl] + AD[b][sl]
                SEX[b][sl] = jnp.exp((_lrelu(z) - CV[b][sl]) * ktinv)
                SIDX[b][sl] = IS[b][sl]

        def issue_scat(b, blk):
            pltpu.async_copy(SEX[b], sh_t.at[SIDX[b]], SW1[b], add=True)
            if write_p:
                base = tile_base + blk * B
                pltpu.async_copy(SEX[b], p_hbm.at[pl.ds(base, B)], SW2[b])

        def wait_scat(b, blk):
            pltpu.make_async_copy(SEX[b], sh_t.at[SIDX[b]], SW1[b]).wait()
            if write_p:
                base = tile_base + blk * B
                pltpu.make_async_copy(SEX[b], p_hbm.at[pl.ds(base, B)],
                                      SW2[b]).wait()

        issue_idx(0, 0)
        wait_idx(0, 0)
        issue_gath(0)
        issue_idx(1, 1)

        def outer(g, carry):
            for b in (0, 1):
                blk = g * 2 + b
                o = 1 - b
                wait_gath(b)

                @pl.when(blk < NBLK - 1)
                def _():
                    wait_idx(o, blk + 1)
                    issue_gath(o)

                @pl.when(blk >= 2)
                def _():
                    wait_scat(b, blk - 2)

                compute(b)
                issue_scat(b, blk)

                @pl.when(blk < NBLK - 2)
                def _():
                    issue_idx(b, blk + 2)

            return carry

        lax.fori_loop(0, NBLK // 2, outer, 0)
        wait_scat(0, NBLK - 2)
        wait_scat(1, NBLK - 1)
        plsc.subcore_barrier()

        def wb(k, carry):
            sl = pl.ds(row0 + k * B, B)
            pltpu.sync_copy(sh_t.at[sl], SEX[0])
            pltpu.sync_copy(SEX[0], tp_hbm.at[cid, sl])
            return carry

        lax.fori_loop(0, RCH, wb, 0)
        tsl = pl.ds(row0 + RCH * B, TAIL)
        pltpu.sync_copy(sh_t.at[tsl], SEX[0].at[pl.ds(0, TAIL)])
        pltpu.sync_copy(SEX[0].at[pl.ds(0, TAIL)], tp_hbm.at[cid, tsl])

    return lse_kernel


_sc_lse1 = _make_lse(False)
_sc_lse2 = _make_lse(True)


@functools.partial(
    pl.kernel,
    out_type=(
        jax.ShapeDtypeStruct((NC, NPAD, F), jnp.float32),
        jax.ShapeDtypeStruct((NC, NPAD), jnp.float32),
    ),
    mesh=_MESH,
    compiler_params=pltpu.CompilerParams(use_tc_tiling_on_sc=False),
    scratch_types=[
        pltpu.VMEM_SHARED((NPAD, F), jnp.float32),
        pltpu.VMEM_SHARED((NPAD,), jnp.float32),
        pltpu.VMEM_SHARED((NPAD,), jnp.float32),
        [pltpu.VMEM((B,), jnp.int32)] * 2,
        [pltpu.VMEM((B,), jnp.int32)] * 2,
        [pltpu.VMEM((B,), jnp.float32)] * 2,
        [pltpu.VMEM((B,), jnp.float32)] * 2,
        [pltpu.VMEM((B, F), jnp.float32)] * 2,
        [pltpu.VMEM((B,), jnp.int32)] * 2,
        [pltpu.VMEM((B,), jnp.float32)] * 2,
        [pltpu.VMEM((B, F), jnp.float32)] * 2,
        [pltpu.SemaphoreType.DMA] * 2,
        [pltpu.SemaphoreType.DMA] * 2,
        [pltpu.SemaphoreType.DMA] * 2,
        [pltpu.SemaphoreType.DMA] * 2,
        [pltpu.SemaphoreType.DMA] * 2,
        [pltpu.SemaphoreType.DMA] * 2,
        [pltpu.SemaphoreType.DMA] * 2,
    ],
)
def _sc_full(src_hbm, dst_hbm, h_hbm, p_hbm, w4_hbm,
             outp_hbm, sp_hbm,
             sh_out, sh_s, sh_w4,
             IS, ID, PV, WV, ROWS, SIDX, SEX, SROWS,
             SIS, SID, SP, SG1, SG2, SW1, SW2):
    cid = lax.axis_index("c")
    sid = lax.axis_index("s")
    row0 = sid * ROWS_PT

    _zero_vmem(SROWS[0], B)
    zero16 = jnp.zeros((LN,), jnp.float32)
    for j in range(B // LN):
        SEX[0][pl.ds(j * LN, LN)] = zero16

    def zchunk(k, carry):
        sl = pl.ds(row0 + k * B, B)
        pltpu.sync_copy(SROWS[0], sh_out.at[sl])
        pltpu.sync_copy(SEX[0], sh_s.at[sl])
        return carry

    lax.fori_loop(0, RCH, zchunk, 0)
    tsl = pl.ds(row0 + RCH * B, TAIL)
    pltpu.sync_copy(SROWS[0].at[pl.ds(0, TAIL)], sh_out.at[tsl])
    pltpu.sync_copy(SEX[0].at[pl.ds(0, TAIL)], sh_s.at[tsl])
    _stage_scalar(w4_hbm, sh_w4, WV[0], row0)
    plsc.subcore_barrier()

    tile_base = (cid * NS + sid) * TB

    def issue_idx(b, blk):
        base = tile_base + blk * B
        pltpu.async_copy(src_hbm.at[pl.ds(base, B)], IS[b], SIS[b])
        pltpu.async_copy(dst_hbm.at[pl.ds(base, B)], ID[b], SID[b])
        pltpu.async_copy(p_hbm.at[pl.ds(base, B)], PV[b], SP[b])

    def wait_idx(b, blk):
        base = tile_base + blk * B
        pltpu.make_async_copy(src_hbm.at[pl.ds(base, B)], IS[b],
                              SIS[b]).wait()
        pltpu.make_async_copy(dst_hbm.at[pl.ds(base, B)], ID[b],
                              SID[b]).wait()
        pltpu.make_async_copy(p_hbm.at[pl.ds(base, B)], PV[b], SP[b]).wait()

    def issue_gath(b):
        pltpu.async_copy(sh_w4.at[IS[b]], WV[b], SG1[b])
        pltpu.async_copy(h_hbm.at[ID[b]], ROWS[b], SG2[b])

    def wait_gath(b):
        pltpu.make_async_copy(sh_w4.at[IS[b]], WV[b], SG1[b]).wait()
        pltpu.make_async_copy(h_hbm.at[ID[b]], ROWS[b], SG2[b]).wait()

    def compute(b):
        for j in range(B // LN):
            sl = pl.ds(j * LN, LN)
            q = PV[b][sl] * WV[b][sl]
            q2 = q * q
            ex = q2 * q2
            SEX[b][sl] = ex
            SIDX[b][sl] = IS[b][sl]
            for t in range(LN):
                r = j * LN + t
                SROWS[b][r, :] = ROWS[b][r, :] * ex[t]

    def issue_scat(b):
        pltpu.async_copy(SEX[b], sh_s.at[SIDX[b]], SW1[b], add=True)
        pltpu.async_copy(SROWS[b], sh_out.at[SIDX[b]], SW2[b], add=True)

    def wait_scat(b):
        pltpu.make_async_copy(SEX[b], sh_s.at[SIDX[b]], SW1[b]).wait()
        pltpu.make_async_copy(SROWS[b], sh_out.at[SIDX[b]], SW2[b]).wait()

    issue_idx(0, 0)
    wait_idx(0, 0)
    issue_gath(0)
    issue_idx(1, 1)

    def outer(g, carry):
        for b in (0, 1):
            blk = g * 2 + b
            o = 1 - b
            wait_gath(b)

            @pl.when(blk < NBLK - 1)
            def _():
                wait_idx(o, blk + 1)
                issue_gath(o)

            @pl.when(blk >= 2)
            def _():
                wait_scat(b)

            compute(b)
            issue_scat(b)

            @pl.when(blk < NBLK - 2)
            def _():
                issue_idx(b, blk + 2)

        return carry

    lax.fori_loop(0, NBLK // 2, outer, 0)
    wait_scat(0)
    wait_scat(1)
    plsc.subcore_barrier()

    def wb(k, carry):
        sl = pl.ds(row0 + k * B, B)
        pltpu.sync_copy(sh_out.at[sl], ROWS[0])
        pltpu.sync_copy(ROWS[0], outp_hbm.at[cid, sl])
        pltpu.sync_copy(sh_s.at[sl], SEX[0])
        pltpu.sync_copy(SEX[0], sp_hbm.at[cid, sl])
        return carry

    lax.fori_loop(0, RCH, wb, 0)
    tsl = pl.ds(row0 + RCH * B, TAIL)
    pltpu.sync_copy(sh_out.at[tsl], ROWS[0].at[pl.ds(0, TAIL)])
    pltpu.sync_copy(ROWS[0].at[pl.ds(0, TAIL)], outp_hbm.at[cid, tsl])
    pltpu.sync_copy(sh_s.at[tsl], SEX[0].at[pl.ds(0, TAIL)])
    pltpu.sync_copy(SEX[0].at[pl.ds(0, TAIL)], sp_hbm.at[cid, tsl])


def _recip_body(t0_ref, t1_ref, out_ref):
    t = t0_ref[...] + t1_ref[...]
    out_ref[...] = jnp.where(t > 0.0, 1.0 / t, 1.0)


def _recip(t0, t1):
    return pl.pallas_call(
        _recip_body,
        grid=(N // R,),
        in_specs=[
            pl.BlockSpec((R, 1), lambda i: (i, 0)),
            pl.BlockSpec((R, 1), lambda i: (i, 0)),
        ],
        out_specs=pl.BlockSpec((R, 1), lambda i: (i, 0)),
        out_shape=jax.ShapeDtypeStruct((N, 1), jnp.float32),
        compiler_params=pltpu.CompilerParams(
            dimension_semantics=("arbitrary",)),
    )(t0, t1)


def _layer_edge(h, sc, st, srcp, dstp):
    a_s = sc[:, 0]
    a_d = sc[:, 1]
    hi = _lrelu(st[0, 0] + st[0, 2])
    lo = _lrelu(st[0, 1] + st[0, 3])
    kt1 = jnp.maximum(1.0, (hi - lo) * (1.0 / 80.0))

    npad = NPAD - N
    asp = jnp.pad(a_s, (0, npad))
    adp = jnp.pad(a_d, (0, npad))
    c0p = jnp.pad(jnp.full((N,), hi, jnp.float32), (0, npad))

    par1 = jnp.zeros((16,), jnp.float32).at[0].set(1.0 / kt1)
    (tp,) = _sc_lse1(srcp, dstp, asp, c0p, adp, par1)
    c1 = _shift_update(tp[0, :N, None], tp[1, :N, None],
                       c0p[:N, None], kt1[None, None])[:, 0]

    par2 = jnp.zeros((16,), jnp.float32).at[0].set(1.0 / KT2)
    c1p = jnp.pad(c1, (0, npad))
    tp2, pbuf = _sc_lse2(srcp, dstp, asp, c1p, adp, par2)
    w4 = _recip(tp2[0, :N, None], tp2[1, :N, None])[:, 0]
    w4p = jnp.pad(w4, (0, npad))

    hp = jnp.pad(h, ((0, npad), (0, 0)))
    outp, sp = _sc_full(srcp, dstp, hp, pbuf, w4p)
    o0 = outp[0, :N, :]
    o1 = outp[1, :N, :]
    s0 = sp[0, :N, None]
    s1 = sp[1, :N, None]
    return o0, o1, s0, s1


def kernel(local_features, edge_index, W0, a0, W1, a1):
    src = edge_index[0, :]
    dst = edge_index[1, :]
    srcp = jnp.concatenate([src, jnp.full((EPAD - E,), N, jnp.int32)])
    dstp = jnp.concatenate([dst, jnp.full((EPAD - E,), N, jnp.int32)])

    h1, sc1, st1 = _dense1(local_features, W0, a0)
    o0, o1, s0, s1 = _layer_edge(h1, sc1, st1, srcp, dstp)

    h2, sc2, st2 = _dense2(o0, o1, s0, s1, W1, a1)
    o0, o1, s0, s1 = _layer_edge(h2, sc2, st2, srcp, dstp)

    return _final(o0, o1, s0, s1)


# speedup vs baseline: 50.6658x; 1.2185x over previous
import functools

import jax
import jax.numpy as jnp
from jax import lax
from jax.experimental import pallas as pl
from jax.experimental.pallas import tpu as pltpu
from jax.experimental.pallas import tpu_sc as plsc

N = 100000
D = 128
F = 16
E = 3200000

NC, NS, LN = 2, 16, 16
B = 256
BF = 128
NPAD = 100096
ROWS_PT = NPAD // NS
TB = 100352
NBLK = TB // B
NBLKF = TB // BF
EPAD = TB * NC * NS
NEG = 0.01
KT2 = 4.0
R = 2000


def _lrelu(z):
    return jnp.where(z > 0.0, z, NEG * z)


def _dense_math(h, a_ref, i, sc_ref, st_ref):
    a = a_ref[...]
    s_col = jnp.dot(h, a[:F, :], preferred_element_type=jnp.float32)
    d_col = jnp.dot(h, a[F:, :], preferred_element_type=jnp.float32)
    sc_ref[...] = jnp.concatenate([s_col, d_col], axis=1)
    mx_s, mn_s = jnp.max(s_col), jnp.min(s_col)
    mx_d, mn_d = jnp.max(d_col), jnp.min(d_col)

    @pl.when(i == 0)
    def _():
        st_ref[0, 0] = mx_s
        st_ref[0, 1] = mn_s
        st_ref[0, 2] = mx_d
        st_ref[0, 3] = mn_d

    @pl.when(i > 0)
    def _():
        st_ref[0, 0] = jnp.maximum(st_ref[0, 0], mx_s)
        st_ref[0, 1] = jnp.minimum(st_ref[0, 1], mn_s)
        st_ref[0, 2] = jnp.maximum(st_ref[0, 2], mx_d)
        st_ref[0, 3] = jnp.minimum(st_ref[0, 3], mn_d)


def _dense1_body(x_ref, w_ref, a_ref, h_ref, sc_ref, st_ref):
    h = jnp.dot(x_ref[...], w_ref[...], preferred_element_type=jnp.float32)
    h_ref[...] = h
    _dense_math(h, a_ref, pl.program_id(0), sc_ref, st_ref)


def _dense1(x, w, a):
    return pl.pallas_call(
        _dense1_body,
        grid=(N // R,),
        in_specs=[
            pl.BlockSpec((R, D), lambda i: (i, 0)),
            pl.BlockSpec((D, F), lambda i: (0, 0)),
            pl.BlockSpec((2 * F, 1), lambda i: (0, 0)),
        ],
        out_specs=[
            pl.BlockSpec((R, F), lambda i: (i, 0)),
            pl.BlockSpec((R, 2), lambda i: (i, 0)),
            pl.BlockSpec(memory_space=pltpu.SMEM),
        ],
        out_shape=[
            jax.ShapeDtypeStruct((N, F), jnp.float32),
            jax.ShapeDtypeStruct((N, 2), jnp.float32),
            jax.ShapeDtypeStruct((1, 8), jnp.float32),
        ],
        compiler_params=pltpu.CompilerParams(
            dimension_semantics=("arbitrary",)),
    )(x, w, a)


def _dense2_body(o0_ref, o1_ref, s0_ref, s1_ref, w_ref, a_ref,
                 h_ref, sc_ref, st_ref):
    num = o0_ref[...] + o1_ref[...]
    s = s0_ref[...] + s1_ref[...]
    x = jnp.where(s > 0.0, num / s, 0.0)
    x = jnp.where(x > 0.0, x, jnp.exp(jnp.minimum(x, 0.0)) - 1.0)
    h = jnp.dot(x, w_ref[...], preferred_element_type=jnp.float32)
    h_ref[...] = h
    _dense_math(h, a_ref, pl.program_id(0), sc_ref, st_ref)


def _dense2(o0, o1, s0, s1, w, a):
    return pl.pallas_call(
        _dense2_body,
        grid=(N // R,),
        in_specs=[
            pl.BlockSpec((R, F), lambda i: (i, 0)),
            pl.BlockSpec((R, F), lambda i: (i, 0)),
            pl.BlockSpec((R, 1), lambda i: (i, 0)),
            pl.BlockSpec((R, 1), lambda i: (i, 0)),
            pl.BlockSpec((F, F), lambda i: (0, 0)),
            pl.BlockSpec((2 * F, 1), lambda i: (0, 0)),
        ],
        out_specs=[
            pl.BlockSpec((R, F), lambda i: (i, 0)),
            pl.BlockSpec((R, 2), lambda i: (i, 0)),
            pl.BlockSpec(memory_space=pltpu.SMEM),
        ],
        out_shape=[
            jax.ShapeDtypeStruct((N, F), jnp.float32),
            jax.ShapeDtypeStruct((N, 2), jnp.float32),
            jax.ShapeDtypeStruct((1, 8), jnp.float32),
        ],
        compiler_params=pltpu.CompilerParams(
            dimension_semantics=("arbitrary",)),
    )(o0, o1, s0, s1, w, a)


def _shift_body(t0_ref, t1_ref, c_ref, kt_ref, out_ref):
    t = t0_ref[...] + t1_ref[...]
    c = c_ref[...]
    out_ref[...] = jnp.where(t > 0.0, c + kt_ref[0, 0] * jnp.log(t), c)


def _shift_update(t0, t1, c, kt):
    return pl.pallas_call(
        _shift_body,
        grid=(N // R,),
        in_specs=[
            pl.BlockSpec((R, 1), lambda i: (i, 0)),
            pl.BlockSpec((R, 1), lambda i: (i, 0)),
            pl.BlockSpec((R, 1), lambda i: (i, 0)),
            pl.BlockSpec((1, 1), lambda i: (0, 0)),
        ],
        out_specs=pl.BlockSpec((R, 1), lambda i: (i, 0)),
        out_shape=jax.ShapeDtypeStruct((N, 1), jnp.float32),
        compiler_params=pltpu.CompilerParams(
            dimension_semantics=("arbitrary",)),
    )(t0, t1, c, kt)


def _final_body(o0_ref, o1_ref, s0_ref, s1_ref, y_ref):
    num = o0_ref[...] + o1_ref[...]
    s = s0_ref[...] + s1_ref[...]
    z = jnp.where(s > 0.0, num / s, 0.0)
    m = jnp.max(z, axis=1, keepdims=True)
    zs = z - m
    y_ref[...] = zs - jnp.log(jnp.sum(jnp.exp(zs), axis=1, keepdims=True))


def _final(o0, o1, s0, s1):
    return pl.pallas_call(
        _final_body,
        grid=(N // R,),
        in_specs=[
            pl.BlockSpec((R, F), lambda i: (i, 0)),
            pl.BlockSpec((R, F), lambda i: (i, 0)),
            pl.BlockSpec((R, 1), lambda i: (i, 0)),
            pl.BlockSpec((R, 1), lambda i: (i, 0)),
        ],
        out_specs=pl.BlockSpec((R, F), lambda i: (i, 0)),
        out_shape=jax.ShapeDtypeStruct((N, F), jnp.float32),
        compiler_params=pltpu.CompilerParams(
            dimension_semantics=("arbitrary",)),
    )(o0, o1, s0, s1)


_MESH = plsc.VectorSubcoreMesh(
    core_axis_name="c", subcore_axis_name="s", num_cores=NC, num_subcores=NS)


def _zero_vmem(buf, n_rows):
    zero16 = jnp.zeros((LN,), jnp.float32)

    def zb(i, carry):
        buf[i, :] = zero16
        return carry

    lax.fori_loop(0, n_rows, zb, 0)


def _stage_scalar(hbm, sh, bounce, row0, bsz):
    rch = ROWS_PT // bsz
    tail = ROWS_PT - rch * bsz

    def body(k, carry):
        sl = pl.ds(row0 + k * bsz, bsz)
        pltpu.sync_copy(hbm.at[sl], bounce)
        pltpu.sync_copy(bounce, sh.at[sl])
        return carry

    lax.fori_loop(0, rch, body, 0)
    sl = pl.ds(row0 + rch * bsz, tail)
    pltpu.sync_copy(hbm.at[sl], bounce.at[pl.ds(0, tail)])
    pltpu.sync_copy(bounce.at[pl.ds(0, tail)], sh.at[sl])


def _zero_shared(sh, zbuf, row0, bsz):
    rch = ROWS_PT // bsz
    tail = ROWS_PT - rch * bsz

    def body(k, carry):
        pltpu.sync_copy(zbuf, sh.at[pl.ds(row0 + k * bsz, bsz)])
        return carry

    lax.fori_loop(0, rch, body, 0)
    pltpu.sync_copy(zbuf.at[pl.ds(0, tail)],
                    sh.at[pl.ds(row0 + rch * bsz, tail)])

def _make_lse(write_p):
    out_type = (jax.ShapeDtypeStruct((NC, NPAD), jnp.float32),)
    if write_p:
        out_type = out_type + (jax.ShapeDtypeStruct((EPAD,), jnp.float32),)

    @functools.partial(
        pl.kernel,
        out_type=out_type,
        mesh=_MESH,
        compiler_params=pltpu.CompilerParams(use_tc_tiling_on_sc=False),
        scratch_types=[
            pltpu.VMEM_SHARED((NPAD,), jnp.float32),
            pltpu.VMEM_SHARED((NPAD,), jnp.float32),
            pltpu.VMEM_SHARED((NPAD,), jnp.float32),
            pltpu.VMEM_SHARED((NPAD,), jnp.float32),
            [pltpu.VMEM((B,), jnp.int32)] * 2,
            [pltpu.VMEM((B,), jnp.int32)] * 2,
            [pltpu.VMEM((B,), jnp.float32)] * 2,
            [pltpu.VMEM((B,), jnp.float32)] * 2,
            [pltpu.VMEM((B,), jnp.float32)] * 2,
            [pltpu.VMEM((B,), jnp.int32)] * 2,
            [pltpu.VMEM((B,), jnp.float32)] * 2,
            [pltpu.SemaphoreType.DMA] * 2,
            [pltpu.SemaphoreType.DMA] * 2,
            [pltpu.SemaphoreType.DMA] * 2,
            [pltpu.SemaphoreType.DMA] * 2,
            [pltpu.SemaphoreType.DMA] * 2,
            [pltpu.SemaphoreType.DMA] * 2,
            [pltpu.SemaphoreType.DMA] * 2,
            pltpu.VMEM((16,), jnp.float32),
        ],
    )
    def lse_kernel(src_hbm, dst_hbm, as_hbm, c_hbm, ad_hbm, par_hbm, *rest):
        if write_p:
            (tp_hbm, p_hbm) = rest[0], rest[1]
            (sh_t, sh_as, sh_c, sh_ad, IS, ID, AS, CV, AD, SIDX, SEX,
             SIS, SID, SG1, SG2, SG3, SW1, SW2, par) = rest[2:]
        else:
            tp_hbm = rest[0]
            p_hbm = None
            (sh_t, sh_as, sh_c, sh_ad, IS, ID, AS, CV, AD, SIDX, SEX,
             SIS, SID, SG1, SG2, SG3, SW1, SW2, par) = rest[1:]

        cid = lax.axis_index("c")
        sid = lax.axis_index("s")
        row0 = sid * ROWS_PT

        pltpu.sync_copy(par_hbm, par)
        ktinv = par[pl.ds(0, LN)][0]

        zero16 = jnp.zeros((LN,), jnp.float32)
        for j in range(B // LN):
            SEX[0][pl.ds(j * LN, LN)] = zero16
        _zero_shared(sh_t, SEX[0], row0, B)
        _stage_scalar(as_hbm, sh_as, AS[0], row0, B)
        _stage_scalar(c_hbm, sh_c, CV[0], row0, B)
        _stage_scalar(ad_hbm, sh_ad, AD[0], row0, B)
        plsc.subcore_barrier()

        tile_base = (cid * NS + sid) * TB

        def issue_idx(b, blk):
            base = tile_base + blk * B
            pltpu.async_copy(src_hbm.at[pl.ds(base, B)], IS[b], SIS[b])
            pltpu.async_copy(dst_hbm.at[pl.ds(base, B)], ID[b], SID[b])

        def wait_idx(b, blk):
            base = tile_base + blk * B
            pltpu.make_async_copy(src_hbm.at[pl.ds(base, B)], IS[b],
                                  SIS[b]).wait()
            pltpu.make_async_copy(dst_hbm.at[pl.ds(base, B)], ID[b],
                                  SID[b]).wait()

        def issue_gath(b):
            pltpu.async_copy(sh_as.at[IS[b]], AS[b], SG1[b])
            pltpu.async_copy(sh_c.at[IS[b]], CV[b], SG2[b])
            pltpu.async_copy(sh_ad.at[ID[b]], AD[b], SG3[b])

        def wait_gath(b):
            pltpu.make_async_copy(sh_as.at[IS[b]], AS[b], SG1[b]).wait()
            pltpu.make_async_copy(sh_c.at[IS[b]], CV[b], SG2[b]).wait()
            pltpu.make_async_copy(sh_ad.at[ID[b]], AD[b], SG3[b]).wait()

        def compute(b):
            for j in range(B // LN):
                sl = pl.ds(j * LN, LN)
                z = AS[b][sl] + AD[b][sl]
                SEX[b][sl] = jnp.exp((_lrelu(z) - CV[b][sl]) * ktinv)
                SIDX[b][sl] = IS[b][sl]

        def issue_scat(b, blk):
            pltpu.async_copy(SEX[b], sh_t.at[SIDX[b]], SW1[b], add=True)
            if write_p:
                base = tile_base + blk * B
                pltpu.async_copy(SEX[b], p_hbm.at[pl.ds(base, B)], SW2[b])

        def wait_scat(b, blk):
            pltpu.make_async_copy(SEX[b], sh_t.at[SIDX[b]], SW1[b]).wait()
            if write_p:
                base = tile_base + blk * B
                pltpu.make_async_copy(SEX[b], p_hbm.at[pl.ds(base, B)],
                                      SW2[b]).wait()

        issue_idx(0, 0)
        wait_idx(0, 0)
        issue_gath(0)
        issue_idx(1, 1)

        def outer(g, carry):
            for b in (0, 1):
                blk = g * 2 + b
                o = 1 - b
                wait_gath(b)

                @pl.when(blk < NBLK - 1)
                def _():
                    wait_idx(o, blk + 1)
                    issue_gath(o)

                @pl.when(blk >= 2)
                def _():
                    wait_scat(b, blk - 2)

                compute(b)
                issue_scat(b, blk)

                @pl.when(blk < NBLK - 2)
                def _():
                    issue_idx(b, blk + 2)

            return carry

        lax.fori_loop(0, NBLK // 2, outer, 0)
        wait_scat(0, NBLK - 2)
        wait_scat(1, NBLK - 1)
        plsc.subcore_barrier()

        rch = ROWS_PT // B
        tail = ROWS_PT - rch * B

        def wb(k, carry):
            sl = pl.ds(row0 + k * B, B)
            pltpu.sync_copy(sh_t.at[sl], SEX[0])
            pltpu.sync_copy(SEX[0], tp_hbm.at[cid, sl])
            return carry

        lax.fori_loop(0, rch, wb, 0)
        tsl = pl.ds(row0 + rch * B, tail)
        pltpu.sync_copy(sh_t.at[tsl], SEX[0].at[pl.ds(0, tail)])
        pltpu.sync_copy(SEX[0].at[pl.ds(0, tail)], tp_hbm.at[cid, tsl])

    return lse_kernel


_sc_lse1 = _make_lse(False)
_sc_lse2 = _make_lse(True)


@functools.partial(
    pl.kernel,
    out_type=(
        jax.ShapeDtypeStruct((NC, NPAD, F), jnp.float32),
        jax.ShapeDtypeStruct((NC, NPAD), jnp.float32),
    ),
    mesh=_MESH,
    compiler_params=pltpu.CompilerParams(use_tc_tiling_on_sc=False),
    scratch_types=[
        pltpu.VMEM_SHARED((NPAD, F), jnp.float32),
        pltpu.VMEM_SHARED((NPAD,), jnp.float32),
        pltpu.VMEM_SHARED((NPAD,), jnp.float32),
        [pltpu.VMEM((BF,), jnp.int32)] * 2,
        [pltpu.VMEM((BF,), jnp.int32)] * 2,
        [pltpu.VMEM((BF,), jnp.float32)] * 2,
        [pltpu.VMEM((BF,), jnp.float32)] * 2,
        [pltpu.VMEM((BF, F), jnp.float32)] * 2,
        [pltpu.VMEM((BF,), jnp.int32)] * 2,
        [pltpu.VMEM((BF,), jnp.float32)] * 2,
        [pltpu.VMEM((BF, F), jnp.float32)] * 2,
        [pltpu.SemaphoreType.DMA] * 2,
        [pltpu.SemaphoreType.DMA] * 2,
        [pltpu.SemaphoreType.DMA] * 2,
        [pltpu.SemaphoreType.DMA] * 2,
        [pltpu.SemaphoreType.DMA] * 2,
        [pltpu.SemaphoreType.DMA] * 2,
        [pltpu.SemaphoreType.DMA] * 2,
    ],
)
def _sc_full(src_hbm, dst_hbm, h_hbm, p_hbm, w4_hbm,
             outp_hbm, sp_hbm,
             sh_out, sh_s, sh_w4,
             IS, ID, PV, WV, ROWS, SIDX, SEX, SROWS,
             SIS, SID, SP, SG1, SG2, SW1, SW2):
    cid = lax.axis_index("c")
    sid = lax.axis_index("s")
    row0 = sid * ROWS_PT

    rch = ROWS_PT // BF
    tail = ROWS_PT - rch * BF

    _zero_vmem(SROWS[0], BF)
    zero16 = jnp.zeros((LN,), jnp.float32)
    for j in range(BF // LN):
        SEX[0][pl.ds(j * LN, LN)] = zero16

    def zchunk(k, carry):
        sl = pl.ds(row0 + k * BF, BF)
        pltpu.sync_copy(SROWS[0], sh_out.at[sl])
        pltpu.sync_copy(SEX[0], sh_s.at[sl])
        return carry

    lax.fori_loop(0, rch, zchunk, 0)
    tsl = pl.ds(row0 + rch * BF, tail)
    pltpu.sync_copy(SROWS[0].at[pl.ds(0, tail)], sh_out.at[tsl])
    pltpu.sync_copy(SEX[0].at[pl.ds(0, tail)], sh_s.at[tsl])
    _stage_scalar(w4_hbm, sh_w4, WV[0], row0, BF)
    plsc.subcore_barrier()

    tile_base = (cid * NS + sid) * TB

    def issue_idx(b, blk):
        base = tile_base + blk * BF
        pltpu.async_copy(src_hbm.at[pl.ds(base, BF)], IS[b], SIS[b])
        pltpu.async_copy(dst_hbm.at[pl.ds(base, BF)], ID[b], SID[b])
        pltpu.async_copy(p_hbm.at[pl.ds(base, BF)], PV[b], SP[b])

    def wait_idx(b, blk):
        base = tile_base + blk * BF
        pltpu.make_async_copy(src_hbm.at[pl.ds(base, BF)], IS[b],
                              SIS[b]).wait()
        pltpu.make_async_copy(dst_hbm.at[pl.ds(base, BF)], ID[b],
                              SID[b]).wait()
        pltpu.make_async_copy(p_hbm.at[pl.ds(base, BF)], PV[b], SP[b]).wait()

    def issue_gath(b):
        pltpu.async_copy(sh_w4.at[IS[b]], WV[b], SG1[b])
        pltpu.async_copy(h_hbm.at[ID[b]], ROWS[b], SG2[b])

    def wait_gath(b):
        pltpu.make_async_copy(sh_w4.at[IS[b]], WV[b], SG1[b]).wait()
        pltpu.make_async_copy(h_hbm.at[ID[b]], ROWS[b], SG2[b]).wait()

    def compute(b):
        for j in range(BF // LN):
            sl = pl.ds(j * LN, LN)
            q = PV[b][sl] * WV[b][sl]
            q2 = q * q
            ex = q2 * q2
            SEX[b][sl] = ex
            SIDX[b][sl] = IS[b][sl]
            for t in range(LN):
                r = j * LN + t
                SROWS[b][r, :] = ROWS[b][r, :] * ex[t]

    def issue_scat(b):
        pltpu.async_copy(SEX[b], sh_s.at[SIDX[b]], SW1[b], add=True)
        pltpu.async_copy(SROWS[b], sh_out.at[SIDX[b]], SW2[b], add=True)

    def wait_scat(b):
        pltpu.make_async_copy(SEX[b], sh_s.at[SIDX[b]], SW1[b]).wait()
        pltpu.make_async_copy(SROWS[b], sh_out.at[SIDX[b]], SW2[b]).wait()

    issue_idx(0, 0)
    wait_idx(0, 0)
    issue_gath(0)
    issue_idx(1, 1)

    def outer(g, carry):
        for b in (0, 1):
            blk = g * 2 + b
            o = 1 - b
            wait_gath(b)

            @pl.when(blk < NBLKF - 1)
            def _():
                wait_idx(o, blk + 1)
                issue_gath(o)

            @pl.when(blk >= 2)
            def _():
                wait_scat(b)

            compute(b)
            issue_scat(b)

            @pl.when(blk < NBLKF - 2)
            def _():
                issue_idx(b, blk + 2)

        return carry

    lax.fori_loop(0, NBLKF // 2, outer, 0)
    wait_scat(0)
    wait_scat(1)
    plsc.subcore_barrier()

    def wb(k, carry):
        sl = pl.ds(row0 + k * BF, BF)
        pltpu.sync_copy(sh_out.at[sl], ROWS[0])
        pltpu.sync_copy(ROWS[0], outp_hbm.at[cid, sl])
        pltpu.sync_copy(sh_s.at[sl], SEX[0])
        pltpu.sync_copy(SEX[0], sp_hbm.at[cid, sl])
        return carry

    lax.fori_loop(0, rch, wb, 0)
    tsl = pl.ds(row0 + rch * BF, tail)
    pltpu.sync_copy(sh_out.at[tsl], ROWS[0].at[pl.ds(0, tail)])
    pltpu.sync_copy(ROWS[0].at[pl.ds(0, tail)], outp_hbm.at[cid, tsl])
    pltpu.sync_copy(sh_s.at[tsl], SEX[0].at[pl.ds(0, tail)])
    pltpu.sync_copy(SEX[0].at[pl.ds(0, tail)], sp_hbm.at[cid, tsl])


def _recip_body(t0_ref, t1_ref, out_ref):
    t = t0_ref[...] + t1_ref[...]
    out_ref[...] = jnp.where(t > 0.0, 1.0 / t, 1.0)


def _recip(t0, t1):
    return pl.pallas_call(
        _recip_body,
        grid=(N // R,),
        in_specs=[
            pl.BlockSpec((R, 1), lambda i: (i, 0)),
            pl.BlockSpec((R, 1), lambda i: (i, 0)),
        ],
        out_specs=pl.BlockSpec((R, 1), lambda i: (i, 0)),
        out_shape=jax.ShapeDtypeStruct((N, 1), jnp.float32),
        compiler_params=pltpu.CompilerParams(
            dimension_semantics=("arbitrary",)),
    )(t0, t1)


def _layer_edge(h, sc, st, srcp, dstp):
    a_s = sc[:, 0]
    a_d = sc[:, 1]
    hi = _lrelu(st[0, 0] + st[0, 2])
    lo = _lrelu(st[0, 1] + st[0, 3])
    kt1 = jnp.maximum(1.0, (hi - lo) * (1.0 / 80.0))

    npad = NPAD - N
    asp = jnp.pad(a_s, (0, npad))
    adp = jnp.pad(a_d, (0, npad))
    c0p = jnp.pad(jnp.full((N,), hi, jnp.float32), (0, npad))

    par1 = jnp.zeros((16,), jnp.float32).at[0].set(1.0 / kt1)
    (tp,) = _sc_lse1(srcp, dstp, asp, c0p, adp, par1)
    c1 = _shift_update(tp[0, :N, None], tp[1, :N, None],
                       c0p[:N, None], kt1[None, None])[:, 0]

    par2 = jnp.zeros((16,), jnp.float32).at[0].set(1.0 / KT2)
    c1p = jnp.pad(c1, (0, npad))
    tp2, pbuf = _sc_lse2(srcp, dstp, asp, c1p, adp, par2)
    w4 = _recip(tp2[0, :N, None], tp2[1, :N, None])[:, 0]
    w4p = jnp.pad(w4, (0, npad))

    hp = jnp.pad(h, ((0, npad), (0, 0)))
    outp, sp = _sc_full(srcp, dstp, hp, pbuf, w4p)
    o0 = outp[0, :N, :]
    o1 = outp[1, :N, :]
    s0 = sp[0, :N, None]
    s1 = sp[1, :N, None]
    return o0, o1, s0, s1


def kernel(local_features, edge_index, W0, a0, W1, a1):
    src = edge_index[0, :]
    dst = edge_index[1, :]
    srcp = jnp.concatenate([src, jnp.full((EPAD - E,), N, jnp.int32)])
    dstp = jnp.concatenate([dst, jnp.full((EPAD - E,), N, jnp.int32)])

    h1, sc1, st1 = _dense1(local_features, W0, a0)
    o0, o1, s0, s1 = _layer_edge(h1, sc1, st1, srcp, dstp)

    h2, sc2, st2 = _dense2(o0, o1, s0, s1, W1, a1)
    o0, o1, s0, s1 = _layer_edge(h2, sc2, st2, srcp, dstp)

    return _final(o0, o1, s0, s1)


# speedup vs baseline: 56.2981x; 1.1112x over previous
import functools

import jax
import jax.numpy as jnp
from jax import lax
from jax.experimental import pallas as pl
from jax.experimental.pallas import tpu as pltpu
from jax.experimental.pallas import tpu_sc as plsc

N = 100000
D = 128
F = 16
E = 3200000

NC, NS, LN = 2, 16, 16
B = 256
BF = 224
NPAD = 100096
ROWS_PT = NPAD // NS
TB = 100352
NBLK = TB // B
NBLKF = TB // BF
EPAD = TB * NC * NS
NEG = 0.01
KT2 = 4.0
R = 2000


def _lrelu(z):
    return jnp.where(z > 0.0, z, NEG * z)


def _dense_math(h, a_ref, i, sc_ref, st_ref):
    a = a_ref[...]
    s_col = jnp.dot(h, a[:F, :], preferred_element_type=jnp.float32)
    d_col = jnp.dot(h, a[F:, :], preferred_element_type=jnp.float32)
    sc_ref[...] = jnp.concatenate([s_col, d_col], axis=1)
    mx_s, mn_s = jnp.max(s_col), jnp.min(s_col)
    mx_d, mn_d = jnp.max(d_col), jnp.min(d_col)

    @pl.when(i == 0)
    def _():
        st_ref[0, 0] = mx_s
        st_ref[0, 1] = mn_s
        st_ref[0, 2] = mx_d
        st_ref[0, 3] = mn_d

    @pl.when(i > 0)
    def _():
        st_ref[0, 0] = jnp.maximum(st_ref[0, 0], mx_s)
        st_ref[0, 1] = jnp.minimum(st_ref[0, 1], mn_s)
        st_ref[0, 2] = jnp.maximum(st_ref[0, 2], mx_d)
        st_ref[0, 3] = jnp.minimum(st_ref[0, 3], mn_d)


def _dense1_body(x_ref, w_ref, a_ref, h_ref, sc_ref, st_ref):
    h = jnp.dot(x_ref[...], w_ref[...], preferred_element_type=jnp.float32)
    h_ref[...] = h
    _dense_math(h, a_ref, pl.program_id(0), sc_ref, st_ref)


def _dense1(x, w, a):
    return pl.pallas_call(
        _dense1_body,
        grid=(N // R,),
        in_specs=[
            pl.BlockSpec((R, D), lambda i: (i, 0)),
            pl.BlockSpec((D, F), lambda i: (0, 0)),
            pl.BlockSpec((2 * F, 1), lambda i: (0, 0)),
        ],
        out_specs=[
            pl.BlockSpec((R, F), lambda i: (i, 0)),
            pl.BlockSpec((R, 2), lambda i: (i, 0)),
            pl.BlockSpec(memory_space=pltpu.SMEM),
        ],
        out_shape=[
            jax.ShapeDtypeStruct((N, F), jnp.float32),
            jax.ShapeDtypeStruct((N, 2), jnp.float32),
            jax.ShapeDtypeStruct((1, 8), jnp.float32),
        ],
        compiler_params=pltpu.CompilerParams(
            dimension_semantics=("arbitrary",)),
    )(x, w, a)


def _dense2_body(o0_ref, o1_ref, s0_ref, s1_ref, w_ref, a_ref,
                 h_ref, sc_ref, st_ref):
    num = o0_ref[...] + o1_ref[...]
    s = s0_ref[...] + s1_ref[...]
    x = jnp.where(s > 0.0, num / s, 0.0)
    x = jnp.where(x > 0.0, x, jnp.exp(jnp.minimum(x, 0.0)) - 1.0)
    h = jnp.dot(x, w_ref[...], preferred_element_type=jnp.float32)
    h_ref[...] = h
    _dense_math(h, a_ref, pl.program_id(0), sc_ref, st_ref)


def _dense2(o0, o1, s0, s1, w, a):
    return pl.pallas_call(
        _dense2_body,
        grid=(N // R,),
        in_specs=[
            pl.BlockSpec((R, F), lambda i: (i, 0)),
            pl.BlockSpec((R, F), lambda i: (i, 0)),
            pl.BlockSpec((R, 1), lambda i: (i, 0)),
            pl.BlockSpec((R, 1), lambda i: (i, 0)),
            pl.BlockSpec((F, F), lambda i: (0, 0)),
            pl.BlockSpec((2 * F, 1), lambda i: (0, 0)),
        ],
        out_specs=[
            pl.BlockSpec((R, F), lambda i: (i, 0)),
            pl.BlockSpec((R, 2), lambda i: (i, 0)),
            pl.BlockSpec(memory_space=pltpu.SMEM),
        ],
        out_shape=[
            jax.ShapeDtypeStruct((N, F), jnp.float32),
            jax.ShapeDtypeStruct((N, 2), jnp.float32),
            jax.ShapeDtypeStruct((1, 8), jnp.float32),
        ],
        compiler_params=pltpu.CompilerParams(
            dimension_semantics=("arbitrary",)),
    )(o0, o1, s0, s1, w, a)


def _shift_body(t0_ref, t1_ref, c_ref, kt_ref, out_ref):
    t = t0_ref[...] + t1_ref[...]
    c = c_ref[...]
    out_ref[...] = jnp.where(t > 0.0, c + kt_ref[0, 0] * jnp.log(t), c)


def _shift_update(t0, t1, c, kt):
    return pl.pallas_call(
        _shift_body,
        grid=(N // R,),
        in_specs=[
            pl.BlockSpec((R, 1), lambda i: (i, 0)),
            pl.BlockSpec((R, 1), lambda i: (i, 0)),
            pl.BlockSpec((R, 1), lambda i: (i, 0)),
            pl.BlockSpec((1, 1), lambda i: (0, 0)),
        ],
        out_specs=pl.BlockSpec((R, 1), lambda i: (i, 0)),
        out_shape=jax.ShapeDtypeStruct((N, 1), jnp.float32),
        compiler_params=pltpu.CompilerParams(
            dimension_semantics=("arbitrary",)),
    )(t0, t1, c, kt)


def _final_body(o0_ref, o1_ref, s0_ref, s1_ref, y_ref):
    num = o0_ref[...] + o1_ref[...]
    s = s0_ref[...] + s1_ref[...]
    z = jnp.where(s > 0.0, num / s, 0.0)
    m = jnp.max(z, axis=1, keepdims=True)
    zs = z - m
    y_ref[...] = zs - jnp.log(jnp.sum(jnp.exp(zs), axis=1, keepdims=True))


def _final(o0, o1, s0, s1):
    return pl.pallas_call(
        _final_body,
        grid=(N // R,),
        in_specs=[
            pl.BlockSpec((R, F), lambda i: (i, 0)),
            pl.BlockSpec((R, F), lambda i: (i, 0)),
            pl.BlockSpec((R, 1), lambda i: (i, 0)),
            pl.BlockSpec((R, 1), lambda i: (i, 0)),
        ],
        out_specs=pl.BlockSpec((R, F), lambda i: (i, 0)),
        out_shape=jax.ShapeDtypeStruct((N, F), jnp.float32),
        compiler_params=pltpu.CompilerParams(
            dimension_semantics=("arbitrary",)),
    )(o0, o1, s0, s1)


_MESH = plsc.VectorSubcoreMesh(
    core_axis_name="c", subcore_axis_name="s", num_cores=NC, num_subcores=NS)


def _zero_vmem(buf, n_rows):
    zero16 = jnp.zeros((LN,), jnp.float32)

    def zb(i, carry):
        buf[i, :] = zero16
        return carry

    lax.fori_loop(0, n_rows, zb, 0)


def _stage_scalar(hbm, sh, bounce, row0, bsz):
    rch = ROWS_PT // bsz
    tail = ROWS_PT - rch * bsz

    def body(k, carry):
        sl = pl.ds(row0 + k * bsz, bsz)
        pltpu.sync_copy(hbm.at[sl], bounce)
        pltpu.sync_copy(bounce, sh.at[sl])
        return carry

    lax.fori_loop(0, rch, body, 0)
    sl = pl.ds(row0 + rch * bsz, tail)
    pltpu.sync_copy(hbm.at[sl], bounce.at[pl.ds(0, tail)])
    pltpu.sync_copy(bounce.at[pl.ds(0, tail)], sh.at[sl])


def _zero_shared(sh, zbuf, row0, bsz):
    rch = ROWS_PT // bsz
    tail = ROWS_PT - rch * bsz

    def body(k, carry):
        pltpu.sync_copy(zbuf, sh.at[pl.ds(row0 + k * bsz, bsz)])
        return carry

    lax.fori_loop(0, rch, body, 0)
    pltpu.sync_copy(zbuf.at[pl.ds(0, tail)],
                    sh.at[pl.ds(row0 + rch * bsz, tail)])

def _make_lse(write_p):
    out_type = (jax.ShapeDtypeStruct((NC, NPAD), jnp.float32),)
    if write_p:
        out_type = out_type + (jax.ShapeDtypeStruct((EPAD,), jnp.float32),)

    @functools.partial(
        pl.kernel,
        out_type=out_type,
        mesh=_MESH,
        compiler_params=pltpu.CompilerParams(use_tc_tiling_on_sc=False),
        scratch_types=[
            pltpu.VMEM_SHARED((NPAD,), jnp.float32),
            pltpu.VMEM_SHARED((NPAD,), jnp.float32),
            pltpu.VMEM_SHARED((NPAD,), jnp.float32),
            pltpu.VMEM_SHARED((NPAD,), jnp.float32),
            [pltpu.VMEM((B,), jnp.int32)] * 2,
            [pltpu.VMEM((B,), jnp.int32)] * 2,
            [pltpu.VMEM((B,), jnp.float32)] * 2,
            [pltpu.VMEM((B,), jnp.float32)] * 2,
            [pltpu.VMEM((B,), jnp.float32)] * 2,
            [pltpu.VMEM((B,), jnp.int32)] * 2,
            [pltpu.VMEM((B,), jnp.float32)] * 2,
            [pltpu.SemaphoreType.DMA] * 2,
            [pltpu.SemaphoreType.DMA] * 2,
            [pltpu.SemaphoreType.DMA] * 2,
            [pltpu.SemaphoreType.DMA] * 2,
            [pltpu.SemaphoreType.DMA] * 2,
            [pltpu.SemaphoreType.DMA] * 2,
            [pltpu.SemaphoreType.DMA] * 2,
            pltpu.VMEM((16,), jnp.float32),
        ],
    )
    def lse_kernel(src_hbm, dst_hbm, as_hbm, c_hbm, ad_hbm, par_hbm, *rest):
        if write_p:
            (tp_hbm, p_hbm) = rest[0], rest[1]
            (sh_t, sh_as, sh_c, sh_ad, IS, ID, AS, CV, AD, SIDX, SEX,
             SIS, SID, SG1, SG2, SG3, SW1, SW2, par) = rest[2:]
        else:
            tp_hbm = rest[0]
            p_hbm = None
            (sh_t, sh_as, sh_c, sh_ad, IS, ID, AS, CV, AD, SIDX, SEX,
             SIS, SID, SG1, SG2, SG3, SW1, SW2, par) = rest[1:]

        cid = lax.axis_index("c")
        sid = lax.axis_index("s")
        row0 = sid * ROWS_PT

        pltpu.sync_copy(par_hbm, par)
        ktinv = par[pl.ds(0, LN)][0]

        zero16 = jnp.zeros((LN,), jnp.float32)
        for j in range(B // LN):
            SEX[0][pl.ds(j * LN, LN)] = zero16
        _zero_shared(sh_t, SEX[0], row0, B)
        _stage_scalar(as_hbm, sh_as, AS[0], row0, B)
        _stage_scalar(c_hbm, sh_c, CV[0], row0, B)
        _stage_scalar(ad_hbm, sh_ad, AD[0], row0, B)
        plsc.subcore_barrier()

        tile_base = (cid * NS + sid) * TB

        def issue_idx(b, blk):
            base = tile_base + blk * B
            pltpu.async_copy(src_hbm.at[pl.ds(base, B)], IS[b], SIS[b])
            pltpu.async_copy(dst_hbm.at[pl.ds(base, B)], ID[b], SID[b])

        def wait_idx(b, blk):
            base = tile_base + blk * B
            pltpu.make_async_copy(src_hbm.at[pl.ds(base, B)], IS[b],
                                  SIS[b]).wait()
            pltpu.make_async_copy(dst_hbm.at[pl.ds(base, B)], ID[b],
                                  SID[b]).wait()

        def issue_gath(b):
            pltpu.async_copy(sh_as.at[IS[b]], AS[b], SG1[b])
            pltpu.async_copy(sh_c.at[IS[b]], CV[b], SG2[b])
            pltpu.async_copy(sh_ad.at[ID[b]], AD[b], SG3[b])

        def wait_gath(b):
            pltpu.make_async_copy(sh_as.at[IS[b]], AS[b], SG1[b]).wait()
            pltpu.make_async_copy(sh_c.at[IS[b]], CV[b], SG2[b]).wait()
            pltpu.make_async_copy(sh_ad.at[ID[b]], AD[b], SG3[b]).wait()

        def compute(b):
            for j in range(B // LN):
                sl = pl.ds(j * LN, LN)
                z = AS[b][sl] + AD[b][sl]
                SEX[b][sl] = jnp.exp((_lrelu(z) - CV[b][sl]) * ktinv)
                SIDX[b][sl] = IS[b][sl]

        def issue_scat(b, blk):
            pltpu.async_copy(SEX[b], sh_t.at[SIDX[b]], SW1[b], add=True)
            if write_p:
                base = tile_base + blk * B
                pltpu.async_copy(SEX[b], p_hbm.at[pl.ds(base, B)], SW2[b])

        def wait_scat(b, blk):
            pltpu.make_async_copy(SEX[b], sh_t.at[SIDX[b]], SW1[b]).wait()
            if write_p:
                base = tile_base + blk * B
                pltpu.make_async_copy(SEX[b], p_hbm.at[pl.ds(base, B)],
                                      SW2[b]).wait()

        issue_idx(0, 0)
        wait_idx(0, 0)
        issue_gath(0)
        issue_idx(1, 1)

        def outer(g, carry):
            for b in (0, 1):
                blk = g * 2 + b
                o = 1 - b
                wait_gath(b)

                @pl.when(blk < NBLK - 1)
                def _():
                    wait_idx(o, blk + 1)
                    issue_gath(o)

                @pl.when(blk >= 2)
                def _():
                    wait_scat(b, blk - 2)

                compute(b)
                issue_scat(b, blk)

                @pl.when(blk < NBLK - 2)
                def _():
                    issue_idx(b, blk + 2)

            return carry

        lax.fori_loop(0, NBLK // 2, outer, 0)
        wait_scat(0, NBLK - 2)
        wait_scat(1, NBLK - 1)
        plsc.subcore_barrier()

        rch = ROWS_PT // B
        tail = ROWS_PT - rch * B

        def wb(k, carry):
            sl = pl.ds(row0 + k * B, B)
            pltpu.sync_copy(sh_t.at[sl], SEX[0])
            pltpu.sync_copy(SEX[0], tp_hbm.at[cid, sl])
            return carry

        lax.fori_loop(0, rch, wb, 0)
        tsl = pl.ds(row0 + rch * B, tail)
        pltpu.sync_copy(sh_t.at[tsl], SEX[0].at[pl.ds(0, tail)])
        pltpu.sync_copy(SEX[0].at[pl.ds(0, tail)], tp_hbm.at[cid, tsl])

    return lse_kernel


_sc_lse1 = _make_lse(False)
_sc_lse2 = _make_lse(True)


@functools.partial(
    pl.kernel,
    out_type=(
        jax.ShapeDtypeStruct((NC, NPAD, F), jnp.float32),
        jax.ShapeDtypeStruct((NC, NPAD), jnp.float32),
    ),
    mesh=_MESH,
    compiler_params=pltpu.CompilerParams(use_tc_tiling_on_sc=False),
    scratch_types=[
        pltpu.VMEM_SHARED((NPAD, F), jnp.float32),
        pltpu.VMEM_SHARED((NPAD,), jnp.float32),
        pltpu.VMEM_SHARED((NPAD,), jnp.float32),
        [pltpu.VMEM((BF,), jnp.int32)] * 2,
        [pltpu.VMEM((BF,), jnp.int32)] * 2,
        [pltpu.VMEM((BF,), jnp.float32)] * 2,
        [pltpu.VMEM((BF,), jnp.float32)] * 2,
        [pltpu.VMEM((BF, F), jnp.float32)] * 2,
        [pltpu.VMEM((BF,), jnp.int32)] * 2,
        [pltpu.VMEM((BF,), jnp.float32)] * 2,
        [pltpu.VMEM((BF, F), jnp.float32)] * 2,
        [pltpu.SemaphoreType.DMA] * 2,
        [pltpu.SemaphoreType.DMA] * 2,
        [pltpu.SemaphoreType.DMA] * 2,
        [pltpu.SemaphoreType.DMA] * 2,
        [pltpu.SemaphoreType.DMA] * 2,
        [pltpu.SemaphoreType.DMA] * 2,
        [pltpu.SemaphoreType.DMA] * 2,
    ],
)
def _sc_full(src_hbm, dst_hbm, h_hbm, p_hbm, w4_hbm,
             outp_hbm, sp_hbm,
             sh_out, sh_s, sh_w4,
             IS, ID, PV, WV, ROWS, SIDX, SEX, SROWS,
             SIS, SID, SP, SG1, SG2, SW1, SW2):
    cid = lax.axis_index("c")
    sid = lax.axis_index("s")
    row0 = sid * ROWS_PT

    rch = ROWS_PT // BF
    tail = ROWS_PT - rch * BF

    _zero_vmem(SROWS[0], BF)
    zero16 = jnp.zeros((LN,), jnp.float32)
    for j in range(BF // LN):
        SEX[0][pl.ds(j * LN, LN)] = zero16

    def zchunk(k, carry):
        sl = pl.ds(row0 + k * BF, BF)
        pltpu.sync_copy(SROWS[0], sh_out.at[sl])
        pltpu.sync_copy(SEX[0], sh_s.at[sl])
        return carry

    lax.fori_loop(0, rch, zchunk, 0)
    tsl = pl.ds(row0 + rch * BF, tail)
    pltpu.sync_copy(SROWS[0].at[pl.ds(0, tail)], sh_out.at[tsl])
    pltpu.sync_copy(SEX[0].at[pl.ds(0, tail)], sh_s.at[tsl])
    _stage_scalar(w4_hbm, sh_w4, WV[0], row0, BF)
    plsc.subcore_barrier()

    tile_base = (cid * NS + sid) * TB

    def issue_idx(b, blk):
        base = tile_base + blk * BF
        pltpu.async_copy(src_hbm.at[pl.ds(base, BF)], IS[b], SIS[b])
        pltpu.async_copy(dst_hbm.at[pl.ds(base, BF)], ID[b], SID[b])
        pltpu.async_copy(p_hbm.at[pl.ds(base, BF)], PV[b], SP[b])

    def wait_idx(b, blk):
        base = tile_base + blk * BF
        pltpu.make_async_copy(src_hbm.at[pl.ds(base, BF)], IS[b],
                              SIS[b]).wait()
        pltpu.make_async_copy(dst_hbm.at[pl.ds(base, BF)], ID[b],
                              SID[b]).wait()
        pltpu.make_async_copy(p_hbm.at[pl.ds(base, BF)], PV[b], SP[b]).wait()

    def issue_gath(b):
        pltpu.async_copy(sh_w4.at[IS[b]], WV[b], SG1[b])
        pltpu.async_copy(h_hbm.at[ID[b]], ROWS[b], SG2[b])

    def wait_gath(b):
        pltpu.make_async_copy(sh_w4.at[IS[b]], WV[b], SG1[b]).wait()
        pltpu.make_async_copy(h_hbm.at[ID[b]], ROWS[b], SG2[b]).wait()

    def compute(b):
        for j in range(BF // LN):
            sl = pl.ds(j * LN, LN)
            q = PV[b][sl] * WV[b][sl]
            q2 = q * q
            ex = q2 * q2
            SEX[b][sl] = ex
            SIDX[b][sl] = IS[b][sl]
            for t in range(LN):
                r = j * LN + t
                SROWS[b][r, :] = ROWS[b][r, :] * ex[t]

    def issue_scat(b):
        pltpu.async_copy(SEX[b], sh_s.at[SIDX[b]], SW1[b], add=True)
        pltpu.async_copy(SROWS[b], sh_out.at[SIDX[b]], SW2[b], add=True)

    def wait_scat(b):
        pltpu.make_async_copy(SEX[b], sh_s.at[SIDX[b]], SW1[b]).wait()
        pltpu.make_async_copy(SROWS[b], sh_out.at[SIDX[b]], SW2[b]).wait()

    issue_idx(0, 0)
    wait_idx(0, 0)
    issue_gath(0)
    issue_idx(1, 1)

    def outer(g, carry):
        for b in (0, 1):
            blk = g * 2 + b
            o = 1 - b
            wait_gath(b)

            @pl.when(blk < NBLKF - 1)
            def _():
                wait_idx(o, blk + 1)
                issue_gath(o)

            @pl.when(blk >= 2)
            def _():
                wait_scat(b)

            compute(b)
            issue_scat(b)

            @pl.when(blk < NBLKF - 2)
            def _():
                issue_idx(b, blk + 2)

        return carry

    lax.fori_loop(0, NBLKF // 2, outer, 0)
    wait_scat(0)
    wait_scat(1)
    plsc.subcore_barrier()

    def wb(k, carry):
        sl = pl.ds(row0 + k * BF, BF)
        pltpu.sync_copy(sh_out.at[sl], ROWS[0])
        pltpu.sync_copy(ROWS[0], outp_hbm.at[cid, sl])
        pltpu.sync_copy(sh_s.at[sl], SEX[0])
        pltpu.sync_copy(SEX[0], sp_hbm.at[cid, sl])
        return carry

    lax.fori_loop(0, rch, wb, 0)
    tsl = pl.ds(row0 + rch * BF, tail)
    pltpu.sync_copy(sh_out.at[tsl], ROWS[0].at[pl.ds(0, tail)])
    pltpu.sync_copy(ROWS[0].at[pl.ds(0, tail)], outp_hbm.at[cid, tsl])
    pltpu.sync_copy(sh_s.at[tsl], SEX[0].at[pl.ds(0, tail)])
    pltpu.sync_copy(SEX[0].at[pl.ds(0, tail)], sp_hbm.at[cid, tsl])


def _recip_body(t0_ref, t1_ref, out_ref):
    t = t0_ref[...] + t1_ref[...]
    out_ref[...] = jnp.where(t > 0.0, 1.0 / t, 1.0)


def _recip(t0, t1):
    return pl.pallas_call(
        _recip_body,
        grid=(N // R,),
        in_specs=[
            pl.BlockSpec((R, 1), lambda i: (i, 0)),
            pl.BlockSpec((R, 1), lambda i: (i, 0)),
        ],
        out_specs=pl.BlockSpec((R, 1), lambda i: (i, 0)),
        out_shape=jax.ShapeDtypeStruct((N, 1), jnp.float32),
        compiler_params=pltpu.CompilerParams(
            dimension_semantics=("arbitrary",)),
    )(t0, t1)


def _layer_edge(h, sc, st, srcp, dstp):
    a_s = sc[:, 0]
    a_d = sc[:, 1]
    hi = _lrelu(st[0, 0] + st[0, 2])
    lo = _lrelu(st[0, 1] + st[0, 3])
    kt1 = jnp.maximum(1.0, (hi - lo) * (1.0 / 80.0))

    npad = NPAD - N
    asp = jnp.pad(a_s, (0, npad))
    adp = jnp.pad(a_d, (0, npad))
    c0p = jnp.pad(jnp.full((N,), hi, jnp.float32), (0, npad))

    par1 = jnp.zeros((16,), jnp.float32).at[0].set(1.0 / kt1)
    (tp,) = _sc_lse1(srcp, dstp, asp, c0p, adp, par1)
    c1 = _shift_update(tp[0, :N, None], tp[1, :N, None],
                       c0p[:N, None], kt1[None, None])[:, 0]

    par2 = jnp.zeros((16,), jnp.float32).at[0].set(1.0 / KT2)
    c1p = jnp.pad(c1, (0, npad))
    tp2, pbuf = _sc_lse2(srcp, dstp, asp, c1p, adp, par2)
    w4 = _recip(tp2[0, :N, None], tp2[1, :N, None])[:, 0]
    w4p = jnp.pad(w4, (0, npad))

    hp = jnp.pad(h, ((0, npad), (0, 0)))
    outp, sp = _sc_full(srcp, dstp, hp, pbuf, w4p)
    o0 = outp[0, :N, :]
    o1 = outp[1, :N, :]
    s0 = sp[0, :N, None]
    s1 = sp[1, :N, None]
    return o0, o1, s0, s1


def kernel(local_features, edge_index, W0, a0, W1, a1):
    src = edge_index[0, :]
    dst = edge_index[1, :]
    srcp = jnp.concatenate([src, jnp.full((EPAD - E,), N, jnp.int32)])
    dstp = jnp.concatenate([dst, jnp.full((EPAD - E,), N, jnp.int32)])

    h1, sc1, st1 = _dense1(local_features, W0, a0)
    o0, o1, s0, s1 = _layer_edge(h1, sc1, st1, srcp, dstp)

    h2, sc2, st2 = _dense2(o0, o1, s0, s1, W1, a1)
    o0, o1, s0, s1 = _layer_edge(h2, sc2, st2, srcp, dstp)

    return _final(o0, o1, s0, s1)


# speedup vs baseline: 58.0333x; 1.0308x over previous
import functools

import jax
import jax.numpy as jnp
from jax import lax
from jax.experimental import pallas as pl
from jax.experimental.pallas import tpu as pltpu
from jax.experimental.pallas import tpu_sc as plsc

N = 100000
D = 128
F = 16
E = 3200000

NC, NS, LN = 2, 16, 16
B = 256
BF = 224
NPAD = 100096
ROWS_PT = NPAD // NS
TB = 100352
NBLK = TB // B
NBLKF = TB // BF
EPAD = TB * NC * NS
NEG = 0.01
KT2 = 4.0
R = 2000


def _lrelu(z):
    return jnp.where(z > 0.0, z, NEG * z)


def _dense_math(h, a_ref, i, sc_ref, st_ref):
    a = a_ref[...]
    s_col = jnp.dot(h, a[:F, :], preferred_element_type=jnp.float32)
    d_col = jnp.dot(h, a[F:, :], preferred_element_type=jnp.float32)
    sc_ref[...] = jnp.concatenate([s_col, d_col], axis=1)
    mx_s, mn_s = jnp.max(s_col), jnp.min(s_col)
    mx_d, mn_d = jnp.max(d_col), jnp.min(d_col)

    @pl.when(i == 0)
    def _():
        st_ref[0, 0] = mx_s
        st_ref[0, 1] = mn_s
        st_ref[0, 2] = mx_d
        st_ref[0, 3] = mn_d

    @pl.when(i > 0)
    def _():
        st_ref[0, 0] = jnp.maximum(st_ref[0, 0], mx_s)
        st_ref[0, 1] = jnp.minimum(st_ref[0, 1], mn_s)
        st_ref[0, 2] = jnp.maximum(st_ref[0, 2], mx_d)
        st_ref[0, 3] = jnp.minimum(st_ref[0, 3], mn_d)


def _dense1_body(x_ref, w_ref, a_ref, h_ref, sc_ref, st_ref):
    h = jnp.dot(x_ref[...], w_ref[...], preferred_element_type=jnp.float32)
    h_ref[...] = h
    _dense_math(h, a_ref, pl.program_id(0), sc_ref, st_ref)


def _dense1(x, w, a):
    return pl.pallas_call(
        _dense1_body,
        grid=(N // R,),
        in_specs=[
            pl.BlockSpec((R, D), lambda i: (i, 0)),
            pl.BlockSpec((D, F), lambda i: (0, 0)),
            pl.BlockSpec((2 * F, 1), lambda i: (0, 0)),
        ],
        out_specs=[
            pl.BlockSpec((R, F), lambda i: (i, 0)),
            pl.BlockSpec((R, 2), lambda i: (i, 0)),
            pl.BlockSpec(memory_space=pltpu.SMEM),
        ],
        out_shape=[
            jax.ShapeDtypeStruct((N, F), jnp.float32),
            jax.ShapeDtypeStruct((N, 2), jnp.float32),
            jax.ShapeDtypeStruct((1, 8), jnp.float32),
        ],
        compiler_params=pltpu.CompilerParams(
            dimension_semantics=("arbitrary",)),
    )(x, w, a)


def _dense2_body(o0_ref, o1_ref, s0_ref, s1_ref, w_ref, a_ref,
                 h_ref, sc_ref, st_ref):
    num = o0_ref[...] + o1_ref[...]
    s = s0_ref[...] + s1_ref[...]
    x = jnp.where(s > 0.0, num / s, 0.0)
    x = jnp.where(x > 0.0, x, jnp.exp(jnp.minimum(x, 0.0)) - 1.0)
    h = jnp.dot(x, w_ref[...], preferred_element_type=jnp.float32)
    h_ref[...] = h
    _dense_math(h, a_ref, pl.program_id(0), sc_ref, st_ref)


def _dense2(o0, o1, s0, s1, w, a):
    return pl.pallas_call(
        _dense2_body,
        grid=(N // R,),
        in_specs=[
            pl.BlockSpec((R, F), lambda i: (i, 0)),
            pl.BlockSpec((R, F), lambda i: (i, 0)),
            pl.BlockSpec((R, 1), lambda i: (i, 0)),
            pl.BlockSpec((R, 1), lambda i: (i, 0)),
            pl.BlockSpec((F, F), lambda i: (0, 0)),
            pl.BlockSpec((2 * F, 1), lambda i: (0, 0)),
        ],
        out_specs=[
            pl.BlockSpec((R, F), lambda i: (i, 0)),
            pl.BlockSpec((R, 2), lambda i: (i, 0)),
            pl.BlockSpec(memory_space=pltpu.SMEM),
        ],
        out_shape=[
            jax.ShapeDtypeStruct((N, F), jnp.float32),
            jax.ShapeDtypeStruct((N, 2), jnp.float32),
            jax.ShapeDtypeStruct((1, 8), jnp.float32),
        ],
        compiler_params=pltpu.CompilerParams(
            dimension_semantics=("arbitrary",)),
    )(o0, o1, s0, s1, w, a)


def _shift_body(t0_ref, t1_ref, c_ref, kt_ref, out_ref):
    t = t0_ref[...] + t1_ref[...]
    c = c_ref[0, 0]
    out_ref[...] = jnp.where(t > 0.0, c + kt_ref[0, 0] * jnp.log(t), c)


def _shift_update(t0, t1, c, kt):
    return pl.pallas_call(
        _shift_body,
        grid=(N // R,),
        in_specs=[
            pl.BlockSpec((R, 1), lambda i: (i, 0)),
            pl.BlockSpec((R, 1), lambda i: (i, 0)),
            pl.BlockSpec((1, 1), lambda i: (0, 0)),
            pl.BlockSpec((1, 1), lambda i: (0, 0)),
        ],
        out_specs=pl.BlockSpec((R, 1), lambda i: (i, 0)),
        out_shape=jax.ShapeDtypeStruct((N, 1), jnp.float32),
        compiler_params=pltpu.CompilerParams(
            dimension_semantics=("arbitrary",)),
    )(t0, t1, c, kt)


def _final_body(o0_ref, o1_ref, s0_ref, s1_ref, y_ref):
    num = o0_ref[...] + o1_ref[...]
    s = s0_ref[...] + s1_ref[...]
    z = jnp.where(s > 0.0, num / s, 0.0)
    m = jnp.max(z, axis=1, keepdims=True)
    zs = z - m
    y_ref[...] = zs - jnp.log(jnp.sum(jnp.exp(zs), axis=1, keepdims=True))


def _final(o0, o1, s0, s1):
    return pl.pallas_call(
        _final_body,
        grid=(N // R,),
        in_specs=[
            pl.BlockSpec((R, F), lambda i: (i, 0)),
            pl.BlockSpec((R, F), lambda i: (i, 0)),
            pl.BlockSpec((R, 1), lambda i: (i, 0)),
            pl.BlockSpec((R, 1), lambda i: (i, 0)),
        ],
        out_specs=pl.BlockSpec((R, F), lambda i: (i, 0)),
        out_shape=jax.ShapeDtypeStruct((N, F), jnp.float32),
        compiler_params=pltpu.CompilerParams(
            dimension_semantics=("arbitrary",)),
    )(o0, o1, s0, s1)


_MESH = plsc.VectorSubcoreMesh(
    core_axis_name="c", subcore_axis_name="s", num_cores=NC, num_subcores=NS)


def _zero_vmem(buf, n_rows):
    zero16 = jnp.zeros((LN,), jnp.float32)

    def zb(i, carry):
        buf[i, :] = zero16
        return carry

    lax.fori_loop(0, n_rows, zb, 0)


def _stage_scalar(hbm, sh, bounce, row0, bsz):
    rch = ROWS_PT // bsz
    tail = ROWS_PT - rch * bsz

    def body(k, carry):
        sl = pl.ds(row0 + k * bsz, bsz)
        pltpu.sync_copy(hbm.at[sl], bounce)
        pltpu.sync_copy(bounce, sh.at[sl])
        return carry

    lax.fori_loop(0, rch, body, 0)
    sl = pl.ds(row0 + rch * bsz, tail)
    pltpu.sync_copy(hbm.at[sl], bounce.at[pl.ds(0, tail)])
    pltpu.sync_copy(bounce.at[pl.ds(0, tail)], sh.at[sl])


def _zero_shared(sh, zbuf, row0, bsz):
    rch = ROWS_PT // bsz
    tail = ROWS_PT - rch * bsz

    def body(k, carry):
        pltpu.sync_copy(zbuf, sh.at[pl.ds(row0 + k * bsz, bsz)])
        return carry

    lax.fori_loop(0, rch, body, 0)
    pltpu.sync_copy(zbuf.at[pl.ds(0, tail)],
                    sh.at[pl.ds(row0 + rch * bsz, tail)])

def _make_lse(write_p, const_c):
    out_type = (jax.ShapeDtypeStruct((NC, NPAD), jnp.float32),)
    if write_p:
        out_type = out_type + (jax.ShapeDtypeStruct((EPAD,), jnp.float32),)

    scratch = [
        pltpu.VMEM_SHARED((NPAD,), jnp.float32),
        pltpu.VMEM_SHARED((NPAD,), jnp.float32),
        pltpu.VMEM_SHARED((NPAD,), jnp.float32),
        pltpu.VMEM_SHARED((NPAD,), jnp.float32),
        [pltpu.VMEM((B,), jnp.int32)] * 2,
        [pltpu.VMEM((B,), jnp.int32)] * 2,
        [pltpu.VMEM((B,), jnp.float32)] * 2,
        [pltpu.VMEM((B,), jnp.float32)] * 2,
        [pltpu.VMEM((B,), jnp.float32)] * 2,
        [pltpu.VMEM((B,), jnp.int32)] * 2,
        [pltpu.VMEM((B,), jnp.float32)] * 2,
        [pltpu.SemaphoreType.DMA] * 2,
        [pltpu.SemaphoreType.DMA] * 2,
        [pltpu.SemaphoreType.DMA] * 2,
        [pltpu.SemaphoreType.DMA] * 2,
        [pltpu.SemaphoreType.DMA] * 2,
        [pltpu.SemaphoreType.DMA] * 2,
        [pltpu.SemaphoreType.DMA] * 2,
        pltpu.VMEM((16,), jnp.float32),
    ]

    @functools.partial(
        pl.kernel,
        out_type=out_type,
        mesh=_MESH,
        compiler_params=pltpu.CompilerParams(use_tc_tiling_on_sc=False),
        scratch_types=scratch,
    )
    def lse_kernel(*refs):
        n_in = 5 if const_c else 6
        if const_c:
            (src_hbm, dst_hbm, as_hbm, ad_hbm, par_hbm) = refs[:n_in]
            c_hbm = None
        else:
            (src_hbm, dst_hbm, as_hbm, c_hbm, ad_hbm, par_hbm) = refs[:n_in]
        rest = refs[n_in:]
        if write_p:
            (tp_hbm, p_hbm) = rest[0], rest[1]
            rest = rest[2:]
        else:
            tp_hbm = rest[0]
            p_hbm = None
            rest = rest[1:]
        (sh_t, sh_as, sh_c, sh_ad, IS, ID, AS, CV, AD, SIDX, SEX,
         SIS, SID, SG1, SG2, SG3, SW1, SW2, par) = rest

        cid = lax.axis_index("c")
        sid = lax.axis_index("s")
        row0 = sid * ROWS_PT

        pltpu.sync_copy(par_hbm, par)
        ktinv = par[pl.ds(0, LN)][0]
        cconst = par[pl.ds(0, LN)][1]

        zero16 = jnp.zeros((LN,), jnp.float32)
        for j in range(B // LN):
            SEX[0][pl.ds(j * LN, LN)] = zero16
        _zero_shared(sh_t, SEX[0], row0, B)
        _stage_scalar(as_hbm, sh_as, AS[0], row0, B)
        if not const_c:
            _stage_scalar(c_hbm, sh_c, CV[0], row0, B)
        _stage_scalar(ad_hbm, sh_ad, AD[0], row0, B)
        plsc.subcore_barrier()

        tile_base = (cid * NS + sid) * TB

        def issue_idx(b, blk):
            base = tile_base + blk * B
            pltpu.async_copy(src_hbm.at[pl.ds(base, B)], IS[b], SIS[b])
            pltpu.async_copy(dst_hbm.at[pl.ds(base, B)], ID[b], SID[b])

        def wait_idx(b, blk):
            base = tile_base + blk * B
            pltpu.make_async_copy(src_hbm.at[pl.ds(base, B)], IS[b],
                                  SIS[b]).wait()
            pltpu.make_async_copy(dst_hbm.at[pl.ds(base, B)], ID[b],
                                  SID[b]).wait()

        def issue_gath(b):
            pltpu.async_copy(sh_as.at[IS[b]], AS[b], SG1[b])
            if not const_c:
                pltpu.async_copy(sh_c.at[IS[b]], CV[b], SG2[b])
            pltpu.async_copy(sh_ad.at[ID[b]], AD[b], SG3[b])

        def wait_gath(b):
            pltpu.make_async_copy(sh_as.at[IS[b]], AS[b], SG1[b]).wait()
            if not const_c:
                pltpu.make_async_copy(sh_c.at[IS[b]], CV[b], SG2[b]).wait()
            pltpu.make_async_copy(sh_ad.at[ID[b]], AD[b], SG3[b]).wait()

        def compute(b):
            for j in range(B // LN):
                sl = pl.ds(j * LN, LN)
                z = AS[b][sl] + AD[b][sl]
                cv = cconst if const_c else CV[b][sl]
                SEX[b][sl] = jnp.exp((_lrelu(z) - cv) * ktinv)
                SIDX[b][sl] = IS[b][sl]

        def issue_scat(b, blk):
            pltpu.async_copy(SEX[b], sh_t.at[SIDX[b]], SW1[b], add=True)
            if write_p:
                base = tile_base + blk * B
                pltpu.async_copy(SEX[b], p_hbm.at[pl.ds(base, B)], SW2[b])

        def wait_scat(b, blk):
            pltpu.make_async_copy(SEX[b], sh_t.at[SIDX[b]], SW1[b]).wait()
            if write_p:
                base = tile_base + blk * B
                pltpu.make_async_copy(SEX[b], p_hbm.at[pl.ds(base, B)],
                                      SW2[b]).wait()

        issue_idx(0, 0)
        wait_idx(0, 0)
        issue_gath(0)
        issue_idx(1, 1)

        def outer(g, carry):
            for b in (0, 1):
                blk = g * 2 + b
                o = 1 - b
                wait_gath(b)

                @pl.when(blk < NBLK - 1)
                def _():
                    wait_idx(o, blk + 1)
                    issue_gath(o)

                @pl.when(blk >= 2)
                def _():
                    wait_scat(b, blk - 2)

                compute(b)
                issue_scat(b, blk)

                @pl.when(blk < NBLK - 2)
                def _():
                    issue_idx(b, blk + 2)

            return carry

        lax.fori_loop(0, NBLK // 2, outer, 0)
        wait_scat(0, NBLK - 2)
        wait_scat(1, NBLK - 1)
        plsc.subcore_barrier()

        rch = ROWS_PT // B
        tail = ROWS_PT - rch * B

        def wb(k, carry):
            sl = pl.ds(row0 + k * B, B)
            pltpu.sync_copy(sh_t.at[sl], SEX[0])
            pltpu.sync_copy(SEX[0], tp_hbm.at[cid, sl])
            return carry

        lax.fori_loop(0, rch, wb, 0)
        tsl = pl.ds(row0 + rch * B, tail)
        pltpu.sync_copy(sh_t.at[tsl], SEX[0].at[pl.ds(0, tail)])
        pltpu.sync_copy(SEX[0].at[pl.ds(0, tail)], tp_hbm.at[cid, tsl])

    return lse_kernel


_sc_lse1 = _make_lse(False, True)
_sc_lse2 = _make_lse(True, False)


@functools.partial(
    pl.kernel,
    out_type=(
        jax.ShapeDtypeStruct((NC, NPAD, F), jnp.float32),
        jax.ShapeDtypeStruct((NC, NPAD), jnp.float32),
    ),
    mesh=_MESH,
    compiler_params=pltpu.CompilerParams(use_tc_tiling_on_sc=False),
    scratch_types=[
        pltpu.VMEM_SHARED((NPAD, F), jnp.float32),
        pltpu.VMEM_SHARED((NPAD,), jnp.float32),
        pltpu.VMEM_SHARED((NPAD,), jnp.float32),
        [pltpu.VMEM((BF,), jnp.int32)] * 2,
        [pltpu.VMEM((BF,), jnp.int32)] * 2,
        [pltpu.VMEM((BF,), jnp.float32)] * 2,
        [pltpu.VMEM((BF,), jnp.float32)] * 2,
        [pltpu.VMEM((BF, F), jnp.float32)] * 2,
        [pltpu.VMEM((BF,), jnp.int32)] * 2,
        [pltpu.VMEM((BF,), jnp.float32)] * 2,
        [pltpu.VMEM((BF, F), jnp.float32)] * 2,
        [pltpu.SemaphoreType.DMA] * 2,
        [pltpu.SemaphoreType.DMA] * 2,
        [pltpu.SemaphoreType.DMA] * 2,
        [pltpu.SemaphoreType.DMA] * 2,
        [pltpu.SemaphoreType.DMA] * 2,
        [pltpu.SemaphoreType.DMA] * 2,
        [pltpu.SemaphoreType.DMA] * 2,
    ],
)
def _sc_full(src_hbm, dst_hbm, h_hbm, p_hbm, w4_hbm,
             outp_hbm, sp_hbm,
             sh_out, sh_s, sh_w4,
             IS, ID, PV, WV, ROWS, SIDX, SEX, SROWS,
             SIS, SID, SP, SG1, SG2, SW1, SW2):
    cid = lax.axis_index("c")
    sid = lax.axis_index("s")
    row0 = sid * ROWS_PT

    rch = ROWS_PT // BF
    tail = ROWS_PT - rch * BF

    _zero_vmem(SROWS[0], BF)
    zero16 = jnp.zeros((LN,), jnp.float32)
    for j in range(BF // LN):
        SEX[0][pl.ds(j * LN, LN)] = zero16

    def zchunk(k, carry):
        sl = pl.ds(row0 + k * BF, BF)
        pltpu.sync_copy(SROWS[0], sh_out.at[sl])
        pltpu.sync_copy(SEX[0], sh_s.at[sl])
        return carry

    lax.fori_loop(0, rch, zchunk, 0)
    tsl = pl.ds(row0 + rch * BF, tail)
    pltpu.sync_copy(SROWS[0].at[pl.ds(0, tail)], sh_out.at[tsl])
    pltpu.sync_copy(SEX[0].at[pl.ds(0, tail)], sh_s.at[tsl])
    _stage_scalar(w4_hbm, sh_w4, WV[0], row0, BF)
    plsc.subcore_barrier()

    tile_base = (cid * NS + sid) * TB

    def issue_idx(b, blk):
        base = tile_base + blk * BF
        pltpu.async_copy(src_hbm.at[pl.ds(base, BF)], IS[b], SIS[b])
        pltpu.async_copy(dst_hbm.at[pl.ds(base, BF)], ID[b], SID[b])
        pltpu.async_copy(p_hbm.at[pl.ds(base, BF)], PV[b], SP[b])

    def wait_idx(b, blk):
        base = tile_base + blk * BF
        pltpu.make_async_copy(src_hbm.at[pl.ds(base, BF)], IS[b],
                              SIS[b]).wait()
        pltpu.make_async_copy(dst_hbm.at[pl.ds(base, BF)], ID[b],
                              SID[b]).wait()
        pltpu.make_async_copy(p_hbm.at[pl.ds(base, BF)], PV[b], SP[b]).wait()

    def issue_gath(b):
        pltpu.async_copy(sh_w4.at[IS[b]], WV[b], SG1[b])
        pltpu.async_copy(h_hbm.at[ID[b]], ROWS[b], SG2[b])

    def wait_gath(b):
        pltpu.make_async_copy(sh_w4.at[IS[b]], WV[b], SG1[b]).wait()
        pltpu.make_async_copy(h_hbm.at[ID[b]], ROWS[b], SG2[b]).wait()

    def compute(b):
        for j in range(BF // LN):
            sl = pl.ds(j * LN, LN)
            q = PV[b][sl] * WV[b][sl]
            q2 = q * q
            ex = q2 * q2
            SEX[b][sl] = ex
            SIDX[b][sl] = IS[b][sl]
            for t in range(LN):
                r = j * LN + t
                SROWS[b][r, :] = ROWS[b][r, :] * ex[t]

    def issue_scat(b):
        pltpu.async_copy(SEX[b], sh_s.at[SIDX[b]], SW1[b], add=True)
        pltpu.async_copy(SROWS[b], sh_out.at[SIDX[b]], SW2[b], add=True)

    def wait_scat(b):
        pltpu.make_async_copy(SEX[b], sh_s.at[SIDX[b]], SW1[b]).wait()
        pltpu.make_async_copy(SROWS[b], sh_out.at[SIDX[b]], SW2[b]).wait()

    issue_idx(0, 0)
    wait_idx(0, 0)
    issue_gath(0)
    issue_idx(1, 1)

    def outer(g, carry):
        for b in (0, 1):
            blk = g * 2 + b
            o = 1 - b
            wait_gath(b)

            @pl.when(blk < NBLKF - 1)
            def _():
                wait_idx(o, blk + 1)
                issue_gath(o)

            @pl.when(blk >= 2)
            def _():
                wait_scat(b)

            compute(b)
            issue_scat(b)

            @pl.when(blk < NBLKF - 2)
            def _():
                issue_idx(b, blk + 2)

        return carry

    lax.fori_loop(0, NBLKF // 2, outer, 0)
    wait_scat(0)
    wait_scat(1)
    plsc.subcore_barrier()

    def wb(k, carry):
        sl = pl.ds(row0 + k * BF, BF)
        pltpu.sync_copy(sh_out.at[sl], ROWS[0])
        pltpu.sync_copy(ROWS[0], outp_hbm.at[cid, sl])
        pltpu.sync_copy(sh_s.at[sl], SEX[0])
        pltpu.sync_copy(SEX[0], sp_hbm.at[cid, sl])
        return carry

    lax.fori_loop(0, rch, wb, 0)
    tsl = pl.ds(row0 + rch * BF, tail)
    pltpu.sync_copy(sh_out.at[tsl], ROWS[0].at[pl.ds(0, tail)])
    pltpu.sync_copy(ROWS[0].at[pl.ds(0, tail)], outp_hbm.at[cid, tsl])
    pltpu.sync_copy(sh_s.at[tsl], SEX[0].at[pl.ds(0, tail)])
    pltpu.sync_copy(SEX[0].at[pl.ds(0, tail)], sp_hbm.at[cid, tsl])


def _recip_body(t0_ref, t1_ref, out_ref):
    t = t0_ref[...] + t1_ref[...]
    out_ref[...] = jnp.where(t > 0.0, 1.0 / t, 1.0)


def _recip(t0, t1):
    return pl.pallas_call(
        _recip_body,
        grid=(N // R,),
        in_specs=[
            pl.BlockSpec((R, 1), lambda i: (i, 0)),
            pl.BlockSpec((R, 1), lambda i: (i, 0)),
        ],
        out_specs=pl.BlockSpec((R, 1), lambda i: (i, 0)),
        out_shape=jax.ShapeDtypeStruct((N, 1), jnp.float32),
        compiler_params=pltpu.CompilerParams(
            dimension_semantics=("arbitrary",)),
    )(t0, t1)


def _layer_edge(h, sc, st, srcp, dstp):
    a_s = sc[:, 0]
    a_d = sc[:, 1]
    hi = _lrelu(st[0, 0] + st[0, 2])
    lo = _lrelu(st[0, 1] + st[0, 3])
    kt1 = jnp.maximum(1.0, (hi - lo) * (1.0 / 80.0))

    npad = NPAD - N
    asp = jnp.pad(a_s, (0, npad))
    adp = jnp.pad(a_d, (0, npad))

    par1 = (jnp.zeros((16,), jnp.float32)
            .at[0].set(1.0 / kt1).at[1].set(hi))
    (tp,) = _sc_lse1(srcp, dstp, asp, adp, par1)
    c1 = _shift_update(tp[0, :N, None], tp[1, :N, None],
                       hi[None, None], kt1[None, None])[:, 0]

    par2 = jnp.zeros((16,), jnp.float32).at[0].set(1.0 / KT2)
    c1p = jnp.pad(c1, (0, npad))
    tp2, pbuf = _sc_lse2(srcp, dstp, asp, c1p, adp, par2)
    w4 = _recip(tp2[0, :N, None], tp2[1, :N, None])[:, 0]
    w4p = jnp.pad(w4, (0, npad))

    hp = jnp.pad(h, ((0, npad), (0, 0)))
    outp, sp = _sc_full(srcp, dstp, hp, pbuf, w4p)
    o0 = outp[0, :N, :]
    o1 = outp[1, :N, :]
    s0 = sp[0, :N, None]
    s1 = sp[1, :N, None]
    return o0, o1, s0, s1


def kernel(local_features, edge_index, W0, a0, W1, a1):
    src = edge_index[0, :]
    dst = edge_index[1, :]
    srcp = jnp.concatenate([src, jnp.full((EPAD - E,), N, jnp.int32)])
    dstp = jnp.concatenate([dst, jnp.full((EPAD - E,), N, jnp.int32)])

    h1, sc1, st1 = _dense1(local_features, W0, a0)
    o0, o1, s0, s1 = _layer_edge(h1, sc1, st1, srcp, dstp)

    h2, sc2, st2 = _dense2(o0, o1, s0, s1, W1, a1)
    o0, o1, s0, s1 = _layer_edge(h2, sc2, st2, srcp, dstp)

    return _final(o0, o1, s0, s1)
